# Initial kernel scaffold; baseline (speedup 1.0000x reference)
import jax
import jax.numpy as jnp
from jax.experimental import pallas as pl


def kernel(x, path_list, W_ih, W_hh, b_ih, b_hh, a):
    raise NotImplementedError("write your pallas kernel here")


# speedup vs baseline: 2.3875x; 2.3875x over previous
import functools

import jax
import jax.numpy as jnp
from jax import lax
from jax.experimental import pallas as pl
from jax.experimental.pallas import tpu as pltpu
from jax.experimental.pallas import tpu_sc as plsc

N = 10000
P = 80000
L = 4
IN_DIM = 128
OUT_DIM = 64
HEADS = 4

NC = 2
NS = 16
NW = NC * NS

HALF = 144
PAY = 2 * HALF

G_ROWS = P * L // NW
G_CHUNK = 400
G_ITERS = G_ROWS // G_CHUNK

def _sc_mesh():
    return plsc.VectorSubcoreMesh(
        core_axis_name="c", subcore_axis_name="s", num_cores=NC, num_subcores=NS)


def _gather_body(x_hbm, idx_hbm, out_hbm, idx_v, rows_v, sem):
    wid = lax.axis_index("s") * NC + lax.axis_index("c")
    start = wid * G_ROWS

    def body(i, _):
        base = start + i * G_CHUNK
        pltpu.sync_copy(idx_hbm.at[pl.ds(base, G_CHUNK)], idx_v)
        pltpu.async_copy(x_hbm.at[idx_v], rows_v, sem).wait()
        pltpu.sync_copy(rows_v, out_hbm.at[pl.ds(base, G_CHUNK)])
        return ()

    lax.fori_loop(0, G_ITERS, body, ())


def _gather(x, idx_flat):
    return pl.kernel(
        _gather_body,
        out_type=jax.ShapeDtypeStruct((P * L, IN_DIM), jnp.float32),
        mesh=_sc_mesh(),
        scratch_types=[
            pltpu.VMEM((G_CHUNK,), jnp.int32),
            pltpu.VMEM((G_CHUNK, IN_DIM), jnp.float32),
            pltpu.SemaphoreType.DMA,
        ],
        compiler_params=pltpu.CompilerParams(use_tc_tiling_on_sc=False),
    )(x, idx_flat)


PB = 1000


def _gru_body(f_ref, wih_ref, whh_ref, bih_ref, bhh_ref, a_ref, out_ref):
    f = f_ref[...]
    wih = wih_ref[...]
    whh = whh_ref[...]
    bih = bih_ref[...]
    bhh = bhh_ref[...]
    h = jnp.zeros((PB, OUT_DIM), jnp.float32)
    for t in range(L):
        xt = f[:, t * IN_DIM:(t + 1) * IN_DIM]
        gi = jnp.dot(xt, wih, preferred_element_type=jnp.float32) + bih
        gh = jnp.dot(h, whh, preferred_element_type=jnp.float32) + bhh
        r = jax.nn.sigmoid(gi[:, :OUT_DIM] + gh[:, :OUT_DIM])
        z = jax.nn.sigmoid(gi[:, OUT_DIM:2 * OUT_DIM] + gh[:, OUT_DIM:2 * OUT_DIM])
        n = jnp.tanh(gi[:, 2 * OUT_DIM:] + r * gh[:, 2 * OUT_DIM:])
        h = (1.0 - z) * n + z * h
    att = jnp.exp(jnp.dot(h, a_ref[...], preferred_element_type=jnp.float32))
    attpad = jnp.concatenate([att, jnp.zeros((PB, 16 - HEADS), jnp.float32)], axis=1)
    out_ref[...] = jnp.concatenate(
        [att[:, 0:1] * h, att[:, 1:2] * h, attpad,
         att[:, 2:3] * h, att[:, 3:4] * h, attpad], axis=1)


def _gru_payload(feats, wihT, whhT, bih2, bhh2, a):
    return pl.pallas_call(
        _gru_body,
        grid=(P // PB,),
        in_specs=[
            pl.BlockSpec((PB, L * IN_DIM), lambda i: (i, 0)),
            pl.BlockSpec((IN_DIM, 3 * OUT_DIM), lambda i: (0, 0)),
            pl.BlockSpec((OUT_DIM, 3 * OUT_DIM), lambda i: (0, 0)),
            pl.BlockSpec((1, 3 * OUT_DIM), lambda i: (0, 0)),
            pl.BlockSpec((1, 3 * OUT_DIM), lambda i: (0, 0)),
            pl.BlockSpec((OUT_DIM, HEADS), lambda i: (0, 0)),
        ],
        out_specs=pl.BlockSpec((PB, PAY), lambda i: (i, 0)),
        out_shape=jax.ShapeDtypeStruct((P, PAY), jnp.float32),
    )(feats, wihT, whhT, bih2, bhh2, a)


S_PATHS = P // NS
S_CHUNK = 200
S_ITERS = S_PATHS // S_CHUNK
N_CHUNKS = N // S_CHUNK


def _scatter_body(pay_hbm, seg_hbm, out_hbm, acc, pbuf, sbuf):
    c = lax.axis_index("c")
    s = lax.axis_index("s")

    def zrow(i, _):
        for j in range(HALF // 16):
            pbuf[i, pl.ds(j * 16, 16)] = jnp.zeros((16,), jnp.float32)
        return ()

    lax.fori_loop(0, S_CHUNK, zrow, ())
    for j in range((N_CHUNKS + NS - 1) // NS):
        chunk = s + j * NS

        @pl.when(chunk < N_CHUNKS)
        def _():
            pltpu.sync_copy(pbuf, acc.at[pl.ds(chunk * S_CHUNK, S_CHUNK)])
    plsc.subcore_barrier()

    def body(i, _):
        base = s * S_PATHS + i * S_CHUNK
        pltpu.sync_copy(seg_hbm.at[pl.ds(base, S_CHUNK)], sbuf)
        pltpu.sync_copy(pay_hbm.at[pl.ds(base, S_CHUNK), pl.ds(c * HALF, HALF)], pbuf)
        pltpu.sync_copy(pbuf, acc.at[sbuf], add=True)
        return ()

    lax.fori_loop(0, S_ITERS, body, ())
    plsc.subcore_barrier()

    for j in range((N_CHUNKS + NS - 1) // NS):
        chunk = s + j * NS

        @pl.when(chunk < N_CHUNKS)
        def _():
            pltpu.sync_copy(
                acc.at[pl.ds(chunk * S_CHUNK, S_CHUNK)],
                out_hbm.at[pl.ds(chunk * S_CHUNK, S_CHUNK), pl.ds(c * HALF, HALF)])


def _scatter(payload, seg):
    return pl.kernel(
        _scatter_body,
        out_type=jax.ShapeDtypeStruct((N, PAY), jnp.float32),
        mesh=_sc_mesh(),
        scratch_types=[
            pltpu.VMEM_SHARED((N, HALF), jnp.float32),
            pltpu.VMEM((S_CHUNK, HALF), jnp.float32),
            pltpu.VMEM((S_CHUNK,), jnp.int32),
        ],
        compiler_params=pltpu.CompilerParams(use_tc_tiling_on_sc=False),
    )(payload, seg)


NB = 2000


def _div_body(nd_ref, out_ref):
    nd = nd_ref[...]
    outs = []
    for c in range(2):
        for hh in range(2):
            den = nd[:, c * HALF + 2 * OUT_DIM + 2 * c + hh:
                     c * HALF + 2 * OUT_DIM + 2 * c + hh + 1]
            num = nd[:, c * HALF + hh * OUT_DIM:c * HALF + (hh + 1) * OUT_DIM]
            recip = jnp.where(den != 0.0, 1.0 / den, 0.0)
            outs.append(num * recip)
    out_ref[...] = jnp.concatenate(outs, axis=1)


def _divide(nd):
    return pl.pallas_call(
        _div_body,
        grid=(N // NB,),
        in_specs=[pl.BlockSpec((NB, PAY), lambda i: (i, 0))],
        out_specs=pl.BlockSpec((NB, HEADS * OUT_DIM), lambda i: (i, 0)),
        out_shape=jax.ShapeDtypeStruct((N, HEADS * OUT_DIM), jnp.float32),
    )(nd)


def kernel(x, path_list, W_ih, W_hh, b_ih, b_hh, a):
    idx_flat = path_list.reshape(-1).astype(jnp.int32)
    seg = path_list[:, L - 1].astype(jnp.int32)
    feats = _gather(x, idx_flat).reshape(P, L * IN_DIM)
    payload = _gru_payload(
        feats, W_ih.T, W_hh.T, b_ih.reshape(1, -1), b_hh.reshape(1, -1), a)
    nd = _scatter(payload, seg)
    return _divide(nd)


# speedup vs baseline: 3.2335x; 1.3543x over previous
import jax
import jax.numpy as jnp
from jax import lax
from jax.experimental import pallas as pl
from jax.experimental.pallas import tpu as pltpu
from jax.experimental.pallas import tpu_sc as plsc

N = 10000
NPAD = 10240
P = 80000
L = 4
IN_DIM = 128
OUT_DIM = 64
HEADS = 4

NC = 2
NS = 16
NW = NC * NS

DROWS = NPAD // 8


def _sc_mesh():
    return plsc.VectorSubcoreMesh(
        core_axis_name="c", subcore_axis_name="s", num_cores=NC, num_subcores=NS)


_SC_PARAMS = pltpu.CompilerParams(use_tc_tiling_on_sc=True)

G_ROWS = P // (NW // L)
G_CHUNK = 400
G_ITERS = G_ROWS // G_CHUNK


def _gather_body(x_hbm, idx_hbm, out_hbm, idx_v, rows_v, sem):
    wid = lax.axis_index("s") * NC + lax.axis_index("c")
    t = wid % L
    start = (wid // L) * G_ROWS

    def body(i, _):
        base = start + i * G_CHUNK
        pltpu.sync_copy(idx_hbm.at[pl.ds(t * P + base, G_CHUNK)], idx_v)
        pltpu.async_copy(x_hbm.at[idx_v], rows_v, sem).wait()
        pltpu.sync_copy(rows_v, out_hbm.at[t, pl.ds(base, G_CHUNK)])
        return ()

    lax.fori_loop(0, G_ITERS, body, ())


def _gather(x, idx_tm):
    return pl.kernel(
        _gather_body,
        out_type=jax.ShapeDtypeStruct((L, P, IN_DIM), jnp.float32),
        mesh=_sc_mesh(),
        scratch_types=[
            pltpu.VMEM((G_CHUNK,), jnp.int32),
            pltpu.VMEM((G_CHUNK, IN_DIM), jnp.float32),
            pltpu.SemaphoreType.DMA,
        ],
        compiler_params=_SC_PARAMS,
    )(x, idx_tm)


PB = 1000


def _gru_body(f_ref, wih_ref, whh_ref, bih_ref, bhh_ref, a_ref, m8_ref,
              payn_ref, payd_ref):
    wih = wih_ref[...]
    whh = whh_ref[...]
    bih = bih_ref[...]
    bhh = bhh_ref[...]
    h = jnp.zeros((PB, OUT_DIM), jnp.float32)
    for t in range(L):
        xt = f_ref[t]
        gi = jnp.dot(xt, wih, preferred_element_type=jnp.float32) + bih
        gh = jnp.dot(h, whh, preferred_element_type=jnp.float32) + bhh
        r = jax.nn.sigmoid(gi[:, :OUT_DIM] + gh[:, :OUT_DIM])
        z = jax.nn.sigmoid(gi[:, OUT_DIM:2 * OUT_DIM] + gh[:, OUT_DIM:2 * OUT_DIM])
        n = jnp.tanh(gi[:, 2 * OUT_DIM:] + r * gh[:, 2 * OUT_DIM:])
        h = (1.0 - z) * n + z * h
    att = jnp.exp(jnp.dot(h, a_ref[...], preferred_element_type=jnp.float32))
    payn_ref[...] = jnp.concatenate(
        [att[:, 0:1] * h, att[:, 1:2] * h, att[:, 2:3] * h, att[:, 3:4] * h], axis=1)
    att16 = jnp.concatenate([att, jnp.zeros((PB, 16 - HEADS), jnp.float32)], axis=1)
    pattern = jnp.tile(att16, (1, 8))
    lane_grp = (lax.broadcasted_iota(jnp.int32, (PB, 128), 1) // 16).astype(jnp.float32)
    mask = (lane_grp == m8_ref[...]).astype(jnp.float32)
    payd_ref[...] = pattern * mask


def _gru_payload(feats, wihT, whhT, bih2, bhh2, a, m8):
    return pl.pallas_call(
        _gru_body,
        grid=(P // PB,),
        in_specs=[
            pl.BlockSpec((L, PB, IN_DIM), lambda i: (0, i, 0)),
            pl.BlockSpec((IN_DIM, 3 * OUT_DIM), lambda i: (0, 0)),
            pl.BlockSpec((OUT_DIM, 3 * OUT_DIM), lambda i: (0, 0)),
            pl.BlockSpec((1, 3 * OUT_DIM), lambda i: (0, 0)),
            pl.BlockSpec((1, 3 * OUT_DIM), lambda i: (0, 0)),
            pl.BlockSpec((OUT_DIM, HEADS), lambda i: (0, 0)),
            pl.BlockSpec((PB, 1), lambda i: (i, 0)),
        ],
        out_specs=[
            pl.BlockSpec((PB, HEADS * OUT_DIM), lambda i: (i, 0)),
            pl.BlockSpec((PB, 128), lambda i: (i, 0)),
        ],
        out_shape=[
            jax.ShapeDtypeStruct((P, HEADS * OUT_DIM), jnp.float32),
            jax.ShapeDtypeStruct((P, 128), jnp.float32),
        ],
    )(feats, wihT, whhT, bih2, bhh2, a, m8)


S_CHUNK = 200
S_ITERS = P // NS // S_CHUNK
D_CHUNKS = P // NC // S_CHUNK
Z_CHUNK = 256
ZN_CHUNKS = NPAD // Z_CHUNK
ZD_CHUNKS = DROWS // Z_CHUNK


def _scatter_body(payn_hbm, payd_hbm, seg_hbm, seg8_hbm, num_hbm, den_hbm,
                  accn, accd, pbuf, sbuf):
    c = lax.axis_index("c")
    s = lax.axis_index("s")

    def zrow(i, _):
        for j in range(128 // 16):
            pbuf[i, pl.ds(j * 16, 16)] = jnp.zeros((16,), jnp.float32)
        return ()

    lax.fori_loop(0, Z_CHUNK, zrow, ())
    for j in range((ZN_CHUNKS + NS - 1) // NS):
        chunk = s + j * NS

        @pl.when(chunk < ZN_CHUNKS)
        def _():
            pltpu.sync_copy(pbuf, accn.at[pl.ds(chunk * Z_CHUNK, Z_CHUNK)])

    @pl.when(s < ZD_CHUNKS)
    def _():
        pltpu.sync_copy(pbuf.at[pl.ds(0, Z_CHUNK)],
                        accd.at[pl.ds(s * Z_CHUNK, Z_CHUNK)])
    plsc.subcore_barrier()

    def nbody(i, _):
        base = s * (P // NS) + i * S_CHUNK
        pltpu.sync_copy(seg_hbm.at[pl.ds(base, S_CHUNK)], sbuf)
        pltpu.sync_copy(payn_hbm.at[pl.ds(base, S_CHUNK), pl.ds(c * 128, 128)], pbuf.at[pl.ds(0, S_CHUNK)])
        pltpu.sync_copy(pbuf.at[pl.ds(0, S_CHUNK)], accn.at[sbuf], add=True)
        return ()

    lax.fori_loop(0, S_ITERS, nbody, ())

    def dbody(i, _):
        chunk = s + i * NS

        @pl.when(chunk < D_CHUNKS)
        def _():
            base = c * (P // NC) + chunk * S_CHUNK
            pltpu.sync_copy(seg8_hbm.at[pl.ds(base, S_CHUNK)], sbuf)
            pltpu.sync_copy(payd_hbm.at[pl.ds(base, S_CHUNK)], pbuf.at[pl.ds(0, S_CHUNK)])
            pltpu.sync_copy(pbuf.at[pl.ds(0, S_CHUNK)], accd.at[sbuf], add=True)
        return ()

    lax.fori_loop(0, (D_CHUNKS + NS - 1) // NS, dbody, ())
    plsc.subcore_barrier()

    for j in range((ZN_CHUNKS + NS - 1) // NS):
        chunk = s + j * NS

        @pl.when(chunk < ZN_CHUNKS)
        def _():
            pltpu.sync_copy(
                accn.at[pl.ds(chunk * Z_CHUNK, Z_CHUNK)],
                num_hbm.at[pl.ds(chunk * Z_CHUNK, Z_CHUNK), pl.ds(c * 128, 128)])

    @pl.when(s < ZD_CHUNKS)
    def _():
        pltpu.sync_copy(
            accd.at[pl.ds(s * Z_CHUNK, Z_CHUNK)],
            den_hbm.at[pl.ds(s * Z_CHUNK, Z_CHUNK), pl.ds(c * 128, 128)])


def _scatter(payn, payd, seg, seg8):
    return pl.kernel(
        _scatter_body,
        out_type=[
            jax.ShapeDtypeStruct((NPAD, 2 * 128), jnp.float32),
            jax.ShapeDtypeStruct((DROWS, 2 * 128), jnp.float32),
        ],
        mesh=_sc_mesh(),
        scratch_types=[
            pltpu.VMEM_SHARED((NPAD, 128), jnp.float32),
            pltpu.VMEM_SHARED((DROWS, 128), jnp.float32),
            pltpu.VMEM((Z_CHUNK, 128), jnp.float32),
            pltpu.VMEM((S_CHUNK,), jnp.int32),
        ],
        compiler_params=_SC_PARAMS,
    )(payn, payd, seg, seg8)


NB = 2048


def _div_body(num_ref, den_ref, out_ref):
    num = num_ref[...]
    den = den_ref[...]
    outs = []
    for h in range(HEADS):
        d = den[:, h:h + 1]
        recip = jnp.where(d != 0.0, 1.0 / d, 0.0)
        outs.append(num[:, h * OUT_DIM:(h + 1) * OUT_DIM] * recip)
    out_ref[...] = jnp.concatenate(outs, axis=1)


def _divide(num, den16):
    return pl.pallas_call(
        _div_body,
        grid=(NPAD // NB,),
        in_specs=[
            pl.BlockSpec((NB, HEADS * OUT_DIM), lambda i: (i, 0)),
            pl.BlockSpec((NB, 16), lambda i: (i, 0)),
        ],
        out_specs=pl.BlockSpec((NB, HEADS * OUT_DIM), lambda i: (i, 0)),
        out_shape=jax.ShapeDtypeStruct((N, HEADS * OUT_DIM), jnp.float32),
    )(num, den16)


def kernel(x, path_list, W_ih, W_hh, b_ih, b_hh, a):
    idx_tm = path_list.T.reshape(-1).astype(jnp.int32)
    seg = path_list[:, L - 1].astype(jnp.int32)
    seg8 = seg // 8
    m8 = (seg % 8).astype(jnp.float32).reshape(P, 1)
    feats = _gather(x, idx_tm)
    payn, payd = _gru_payload(
        feats, W_ih.T, W_hh.T, b_ih.reshape(1, -1), b_hh.reshape(1, -1), a, m8)
    num, den = _scatter(payn, payd, seg, seg8)
    den16 = (den[:, :128] + den[:, 128:]).reshape(NPAD, 16)
    return _divide(num, den16)


# speedup vs baseline: 4.1353x; 1.2789x over previous
import jax
import jax.numpy as jnp
from jax import lax
from jax.experimental import pallas as pl
from jax.experimental.pallas import tpu as pltpu
from jax.experimental.pallas import tpu_sc as plsc

N = 10000
NPAD = 10240
P = 80000
L = 4
IN_DIM = 128
OUT_DIM = 64
HEADS = 4

NC = 2
NS = 16
NW = NC * NS

DROWS = NPAD // 8


def _sc_mesh():
    return plsc.VectorSubcoreMesh(
        core_axis_name="c", subcore_axis_name="s", num_cores=NC, num_subcores=NS)


_SC_PARAMS = pltpu.CompilerParams(use_tc_tiling_on_sc=True)

G_ROWS = P // (NW // L)
G_CHUNK = 400
G_ITERS = G_ROWS // G_CHUNK


def _gather_body(x_hbm, idx_hbm, out_hbm, idx_v, rows_v, sem):
    wid = lax.axis_index("s") * NC + lax.axis_index("c")
    t = wid % L
    start = (wid // L) * G_ROWS

    def body(i, _):
        base = start + i * G_CHUNK
        pltpu.sync_copy(idx_hbm.at[pl.ds(t * P + base, G_CHUNK)], idx_v)
        pltpu.async_copy(x_hbm.at[idx_v], rows_v, sem).wait()
        pltpu.sync_copy(rows_v, out_hbm.at[t, pl.ds(base, G_CHUNK)])
        return ()

    lax.fori_loop(0, G_ITERS, body, ())


def _gather(x, idx_tm):
    return pl.kernel(
        _gather_body,
        out_type=jax.ShapeDtypeStruct((L, P, IN_DIM), jnp.float32),
        mesh=_sc_mesh(),
        scratch_types=[
            pltpu.VMEM((G_CHUNK,), jnp.int32),
            pltpu.VMEM((G_CHUNK, IN_DIM), jnp.float32),
            pltpu.SemaphoreType.DMA,
        ],
        compiler_params=_SC_PARAMS,
    )(x, idx_tm)


PB = 1000


def _gru_body(f_ref, wi_ref, wh_ref, bi_ref, bh_ref, abig_ref, irep_ref,
              apat_ref, m8_ref, payn_ref, payd_ref):
    def dot(u, v):
        return jnp.dot(u, v, preferred_element_type=jnp.float32)

    wi = wi_ref[...]
    wh = wh_ref[...]
    b = bi_ref[...] + bh_ref[...]
    h = jnp.zeros((PB, OUT_DIM), jnp.float32)
    for t in range(L):
        xt = f_ref[t]
        gh = dot(h, wh)
        s = dot(xt, wi) + gh + b
        rz = jax.nn.sigmoid(s[:, :2 * OUT_DIM])
        r = rz[:, :OUT_DIM]
        z = rz[:, OUT_DIM:]
        n = jnp.tanh(s[:, 2 * OUT_DIM:] + (r - 1.0) * (gh[:, 2 * OUT_DIM:] + bh_ref[:, 2 * OUT_DIM:]))
        h = (1.0 - z) * n + z * h
    payn_ref[...] = jnp.exp(dot(h, abig_ref[...])) * dot(h, irep_ref[...])
    lane = lax.broadcasted_iota(jnp.int32, (PB, 128), 1)
    grp = (lane // 16).astype(jnp.float32)
    mask = ((grp == m8_ref[...]) & (lane % 16 < HEADS)).astype(jnp.float32)
    payd_ref[...] = jnp.exp(dot(h, apat_ref[...])) * mask


def _gru_payload(feats, wi, wh, bi, bh, abig, irep, apat, m8):
    return pl.pallas_call(
        _gru_body,
        grid=(P // PB,),
        in_specs=[
            pl.BlockSpec((L, PB, IN_DIM), lambda i: (0, i, 0)),
            pl.BlockSpec((IN_DIM, 3 * OUT_DIM), lambda i: (0, 0)),
            pl.BlockSpec((OUT_DIM, 3 * OUT_DIM), lambda i: (0, 0)),
            pl.BlockSpec((1, 3 * OUT_DIM), lambda i: (0, 0)),
            pl.BlockSpec((1, 3 * OUT_DIM), lambda i: (0, 0)),
            pl.BlockSpec((OUT_DIM, HEADS * OUT_DIM), lambda i: (0, 0)),
            pl.BlockSpec((OUT_DIM, HEADS * OUT_DIM), lambda i: (0, 0)),
            pl.BlockSpec((OUT_DIM, 128), lambda i: (0, 0)),
            pl.BlockSpec((PB, 1), lambda i: (i, 0)),
        ],
        out_specs=[
            pl.BlockSpec((PB, HEADS * OUT_DIM), lambda i: (i, 0)),
            pl.BlockSpec((PB, 128), lambda i: (i, 0)),
        ],
        out_shape=[
            jax.ShapeDtypeStruct((P, HEADS * OUT_DIM), jnp.float32),
            jax.ShapeDtypeStruct((P, 128), jnp.float32),
        ],
    )(feats, wi, wh, bi, bh, abig, irep, apat, m8)


S_CHUNK = 200
S_ITERS = P // NS // S_CHUNK
D_CHUNKS = P // NC // S_CHUNK
Z_CHUNK = 256
ZN_CHUNKS = NPAD // Z_CHUNK
ZD_CHUNKS = DROWS // Z_CHUNK


def _scatter_body(payn_hbm, payd_hbm, seg_hbm, seg8_hbm, num_hbm, den_hbm,
                  accn, accd, pbuf, sbuf):
    c = lax.axis_index("c")
    s = lax.axis_index("s")

    def zrow(i, _):
        for j in range(128 // 16):
            pbuf[i, pl.ds(j * 16, 16)] = jnp.zeros((16,), jnp.float32)
        return ()

    lax.fori_loop(0, Z_CHUNK, zrow, ())
    for j in range((ZN_CHUNKS + NS - 1) // NS):
        chunk = s + j * NS

        @pl.when(chunk < ZN_CHUNKS)
        def _():
            pltpu.sync_copy(pbuf, accn.at[pl.ds(chunk * Z_CHUNK, Z_CHUNK)])

    @pl.when(s < ZD_CHUNKS)
    def _():
        pltpu.sync_copy(pbuf.at[pl.ds(0, Z_CHUNK)],
                        accd.at[pl.ds(s * Z_CHUNK, Z_CHUNK)])
    plsc.subcore_barrier()

    def nbody(i, _):
        base = s * (P // NS) + i * S_CHUNK
        pltpu.sync_copy(seg_hbm.at[pl.ds(base, S_CHUNK)], sbuf)
        pltpu.sync_copy(payn_hbm.at[pl.ds(base, S_CHUNK), pl.ds(c * 128, 128)], pbuf.at[pl.ds(0, S_CHUNK)])
        pltpu.sync_copy(pbuf.at[pl.ds(0, S_CHUNK)], accn.at[sbuf], add=True)
        return ()

    lax.fori_loop(0, S_ITERS, nbody, ())

    def dbody(i, _):
        chunk = s + i * NS

        @pl.when(chunk < D_CHUNKS)
        def _():
            base = c * (P // NC) + chunk * S_CHUNK
            pltpu.sync_copy(seg8_hbm.at[pl.ds(base, S_CHUNK)], sbuf)
            pltpu.sync_copy(payd_hbm.at[pl.ds(base, S_CHUNK)], pbuf.at[pl.ds(0, S_CHUNK)])
            pltpu.sync_copy(pbuf.at[pl.ds(0, S_CHUNK)], accd.at[sbuf], add=True)
        return ()

    lax.fori_loop(0, (D_CHUNKS + NS - 1) // NS, dbody, ())
    plsc.subcore_barrier()

    for j in range((ZN_CHUNKS + NS - 1) // NS):
        chunk = s + j * NS

        @pl.when(chunk < ZN_CHUNKS)
        def _():
            pltpu.sync_copy(
                accn.at[pl.ds(chunk * Z_CHUNK, Z_CHUNK)],
                num_hbm.at[pl.ds(chunk * Z_CHUNK, Z_CHUNK), pl.ds(c * 128, 128)])

    @pl.when(s < ZD_CHUNKS)
    def _():
        pltpu.sync_copy(
            accd.at[pl.ds(s * Z_CHUNK, Z_CHUNK)],
            den_hbm.at[pl.ds(s * Z_CHUNK, Z_CHUNK), pl.ds(c * 128, 128)])


def _scatter(payn, payd, seg, seg8):
    return pl.kernel(
        _scatter_body,
        out_type=[
            jax.ShapeDtypeStruct((NPAD, 2 * 128), jnp.float32),
            jax.ShapeDtypeStruct((DROWS, 2 * 128), jnp.float32),
        ],
        mesh=_sc_mesh(),
        scratch_types=[
            pltpu.VMEM_SHARED((NPAD, 128), jnp.float32),
            pltpu.VMEM_SHARED((DROWS, 128), jnp.float32),
            pltpu.VMEM((Z_CHUNK, 128), jnp.float32),
            pltpu.VMEM((S_CHUNK,), jnp.int32),
        ],
        compiler_params=_SC_PARAMS,
    )(payn, payd, seg, seg8)


NB = 2048


def _div_body(num_ref, den_ref, out_ref):
    num = num_ref[...]
    den = den_ref[...]
    outs = []
    for h in range(HEADS):
        d = den[:, h:h + 1]
        recip = jnp.where(d != 0.0, 1.0 / d, 0.0)
        outs.append(num[:, h * OUT_DIM:(h + 1) * OUT_DIM] * recip)
    out_ref[...] = jnp.concatenate(outs, axis=1)


def _divide(num, den16):
    return pl.pallas_call(
        _div_body,
        grid=(NPAD // NB,),
        in_specs=[
            pl.BlockSpec((NB, HEADS * OUT_DIM), lambda i: (i, 0)),
            pl.BlockSpec((NB, 16), lambda i: (i, 0)),
        ],
        out_specs=pl.BlockSpec((NB, HEADS * OUT_DIM), lambda i: (i, 0)),
        out_shape=jax.ShapeDtypeStruct((N, HEADS * OUT_DIM), jnp.float32),
    )(num, den16)


def kernel(x, path_list, W_ih, W_hh, b_ih, b_hh, a):
    idx_tm = path_list.T.reshape(-1).astype(jnp.int32)
    seg = path_list[:, L - 1].astype(jnp.int32)
    seg8 = seg // 8
    m8 = (seg % 8).astype(jnp.float32).reshape(P, 1)
    abig = jnp.repeat(a, OUT_DIM, axis=1)
    irep = jnp.tile(jnp.eye(OUT_DIM, dtype=jnp.float32), (1, HEADS))
    apat = jnp.tile(
        jnp.concatenate([a, jnp.zeros((OUT_DIM, 16 - HEADS), jnp.float32)],
                        axis=1), (1, 8))
    feats = _gather(x, idx_tm)
    payn, payd = _gru_payload(
        feats, W_ih.T, W_hh.T, b_ih.reshape(1, -1), b_hh.reshape(1, -1),
        abig, irep, apat, m8)
    num, den = _scatter(payn, payd, seg, seg8)
    den16 = (den[:, :128] + den[:, 128:]).reshape(NPAD, 16)
    return _divide(num, den16)


# speedup vs baseline: 4.6169x; 1.1164x over previous
import jax
import jax.numpy as jnp
from jax import lax
from jax.experimental import pallas as pl
from jax.experimental.pallas import tpu as pltpu
from jax.experimental.pallas import tpu_sc as plsc

N = 10000
NPAD = 10240
P = 80000
L = 4
IN_DIM = 128
OUT_DIM = 64
HEADS = 4

NC = 2
NS = 16
NW = NC * NS

DROWS = NPAD // 16


def _sc_mesh():
    return plsc.VectorSubcoreMesh(
        core_axis_name="c", subcore_axis_name="s", num_cores=NC, num_subcores=NS)


_SC_PARAMS = pltpu.CompilerParams(use_tc_tiling_on_sc=True)

G_ROWS = P // (NW // L)
G_CHUNK = 400
G_ITERS = G_ROWS // G_CHUNK


def _gather_body(x_hbm, idx_hbm, out_hbm, idx_v, rows_v, sem):
    wid = lax.axis_index("s") * NC + lax.axis_index("c")
    t = wid % L
    start = (wid // L) * G_ROWS

    def body(i, _):
        base = start + i * G_CHUNK
        pltpu.sync_copy(idx_hbm.at[pl.ds(t * P + base, G_CHUNK)], idx_v)
        pltpu.async_copy(x_hbm.at[idx_v], rows_v, sem).wait()
        pltpu.sync_copy(rows_v, out_hbm.at[t, pl.ds(base, G_CHUNK)])
        return ()

    lax.fori_loop(0, G_ITERS, body, ())


def _gather(x, idx_tm):
    return pl.kernel(
        _gather_body,
        out_type=jax.ShapeDtypeStruct((L, P, IN_DIM), jnp.float32),
        mesh=_sc_mesh(),
        scratch_types=[
            pltpu.VMEM((G_CHUNK,), jnp.int32),
            pltpu.VMEM((G_CHUNK, IN_DIM), jnp.float32),
            pltpu.SemaphoreType.DMA,
        ],
        compiler_params=_SC_PARAMS,
    )(x, idx_tm)


PB = 1000


def _gru_body(f_ref, wi_ref, wh_ref, bi_ref, bh_ref, abig_ref, irep_ref,
              apat_ref, m8_ref, payn_ref, payd_ref):
    def dot(u, v):
        return jnp.dot(u, v, preferred_element_type=jnp.float32)

    wi = wi_ref[...]
    wh = wh_ref[...]
    b = bi_ref[...] + bh_ref[...]
    h = jnp.zeros((PB, OUT_DIM), jnp.float32)
    for t in range(L):
        xt = f_ref[t]
        gh = dot(h, wh)
        s = dot(xt, wi) + gh + b
        rz = jax.nn.sigmoid(s[:, :2 * OUT_DIM])
        r = rz[:, :OUT_DIM]
        z = rz[:, OUT_DIM:]
        n = jnp.tanh(s[:, 2 * OUT_DIM:] + (r - 1.0) * (gh[:, 2 * OUT_DIM:] + bh_ref[:, 2 * OUT_DIM:]))
        h = (1.0 - z) * n + z * h
    payn_ref[...] = jnp.exp(dot(h, abig_ref[...])) * dot(h, irep_ref[...])
    lane = lax.broadcasted_iota(jnp.int32, (PB, 128), 1)
    grp = (lane // 8).astype(jnp.float32)
    mask = ((grp == m8_ref[...]) & (lane % 8 < HEADS)).astype(jnp.float32)
    payd_ref[...] = jnp.exp(dot(h, apat_ref[...])) * mask


def _gru_payload(feats, wi, wh, bi, bh, abig, irep, apat, m8):
    return pl.pallas_call(
        _gru_body,
        grid=(P // PB,),
        in_specs=[
            pl.BlockSpec((L, PB, IN_DIM), lambda i: (0, i, 0)),
            pl.BlockSpec((IN_DIM, 3 * OUT_DIM), lambda i: (0, 0)),
            pl.BlockSpec((OUT_DIM, 3 * OUT_DIM), lambda i: (0, 0)),
            pl.BlockSpec((1, 3 * OUT_DIM), lambda i: (0, 0)),
            pl.BlockSpec((1, 3 * OUT_DIM), lambda i: (0, 0)),
            pl.BlockSpec((OUT_DIM, HEADS * OUT_DIM), lambda i: (0, 0)),
            pl.BlockSpec((OUT_DIM, HEADS * OUT_DIM), lambda i: (0, 0)),
            pl.BlockSpec((OUT_DIM, 128), lambda i: (0, 0)),
            pl.BlockSpec((PB, 1), lambda i: (i, 0)),
        ],
        out_specs=[
            pl.BlockSpec((PB, HEADS * OUT_DIM), lambda i: (i, 0)),
            pl.BlockSpec((PB, 128), lambda i: (i, 0)),
        ],
        out_shape=[
            jax.ShapeDtypeStruct((P, HEADS * OUT_DIM), jnp.float32),
            jax.ShapeDtypeStruct((P, 128), jnp.float32),
        ],
    )(feats, wi, wh, bi, bh, abig, irep, apat, m8)


CH = 160
NCHUNK = P // CH
DCHUNK = P // NC // CH
ZN_CHUNKS = NPAD // CH
ZD_CHUNKS = DROWS // CH


def _scatter_body(payn_hbm, payd_hbm, seg_hbm, seg16_hbm, num_hbm, den_hbm,
                  accn, accd, pbuf0, pbuf1, sbuf0, sbuf1, sem0, sem1):
    c = lax.axis_index("c")
    s = lax.axis_index("s")

    def zrow(i, _):
        for j in range(128 // 16):
            pbuf0[i, pl.ds(j * 16, 16)] = jnp.zeros((16,), jnp.float32)
        return ()

    lax.fori_loop(0, CH, zrow, ())
    for j in range(ZN_CHUNKS // NS):
        pltpu.sync_copy(pbuf0, accn.at[pl.ds((s + j * NS) * CH, CH)])

    @pl.when(s < ZD_CHUNKS)
    def _():
        pltpu.sync_copy(pbuf0, accd.at[pl.ds(s * CH, CH)])
    plsc.subcore_barrier()

    def n_src(j):
        base = (s + j * NS) * CH
        return (seg_hbm.at[pl.ds(base, CH)],
                payn_hbm.at[pl.ds(base, CH), pl.ds(c * 128, 128)])

    def n_start(j, pb, sb, sem):
        @pl.when(s + j * NS < NCHUNK)
        def _():
            sg, pay = n_src(j)
            pltpu.async_copy(sg, sb, sem)
            pltpu.async_copy(pay, pb, sem)

    def n_consume(j, pb, sb, sem):
        @pl.when(s + j * NS < NCHUNK)
        def _():
            sg, pay = n_src(j)
            pltpu.make_async_copy(sg, sb, sem).wait()
            pltpu.make_async_copy(pay, pb, sem).wait()
            pltpu.sync_copy(pb, accn.at[sb], add=True)

    NJ = (NCHUNK + NS - 1) // NS

    n_start(0, pbuf0, sbuf0, sem0)

    def npair(k, _):
        j0 = 2 * k
        n_start(j0 + 1, pbuf1, sbuf1, sem1)
        n_consume(j0, pbuf0, sbuf0, sem0)
        n_start(j0 + 2, pbuf0, sbuf0, sem0)
        n_consume(j0 + 1, pbuf1, sbuf1, sem1)
        return ()

    lax.fori_loop(0, NJ // 2, npair, ())

    def d_src(j):
        base = c * (P // NC) + (s + j * NS) * CH
        return (seg16_hbm.at[pl.ds(base, CH)], payd_hbm.at[pl.ds(base, CH)])

    def d_start(j, pb, sb, sem):
        @pl.when(s + j * NS < DCHUNK)
        def _():
            sg, pay = d_src(j)
            pltpu.async_copy(sg, sb, sem)
            pltpu.async_copy(pay, pb, sem)

    def d_consume(j, pb, sb, sem):
        @pl.when(s + j * NS < DCHUNK)
        def _():
            sg, pay = d_src(j)
            pltpu.make_async_copy(sg, sb, sem).wait()
            pltpu.make_async_copy(pay, pb, sem).wait()
            pltpu.sync_copy(pb, accd.at[sb], add=True)

    NJD = (DCHUNK + NS - 1) // NS

    d_start(0, pbuf0, sbuf0, sem0)

    def dpair(k, _):
        j0 = 2 * k
        d_start(j0 + 1, pbuf1, sbuf1, sem1)
        d_consume(j0, pbuf0, sbuf0, sem0)
        d_start(j0 + 2, pbuf0, sbuf0, sem0)
        d_consume(j0 + 1, pbuf1, sbuf1, sem1)
        return ()

    lax.fori_loop(0, NJD // 2, dpair, ())
    plsc.subcore_barrier()

    for j in range(ZN_CHUNKS // NS):
        base = (s + j * NS) * CH
        pltpu.sync_copy(accn.at[pl.ds(base, CH)],
                        num_hbm.at[pl.ds(base, CH), pl.ds(c * 128, 128)])

    @pl.when(s < ZD_CHUNKS)
    def _():
        pltpu.sync_copy(accd.at[pl.ds(s * CH, CH)],
                        den_hbm.at[pl.ds(s * CH, CH), pl.ds(c * 128, 128)])


def _scatter(payn, payd, seg, seg16):
    return pl.kernel(
        _scatter_body,
        out_type=[
            jax.ShapeDtypeStruct((NPAD, 2 * 128), jnp.float32),
            jax.ShapeDtypeStruct((DROWS, 2 * 128), jnp.float32),
        ],
        mesh=_sc_mesh(),
        scratch_types=[
            pltpu.VMEM_SHARED((NPAD, 128), jnp.float32),
            pltpu.VMEM_SHARED((DROWS, 128), jnp.float32),
            pltpu.VMEM((CH, 128), jnp.float32),
            pltpu.VMEM((CH, 128), jnp.float32),
            pltpu.VMEM((CH,), jnp.int32),
            pltpu.VMEM((CH,), jnp.int32),
            pltpu.SemaphoreType.DMA,
            pltpu.SemaphoreType.DMA,
        ],
        compiler_params=_SC_PARAMS,
    )(payn, payd, seg, seg16)


NB = 2048


def _div_body(num_ref, den_ref, out_ref):
    num = num_ref[...]
    den = den_ref[...]
    outs = []
    for h in range(HEADS):
        d = den[:, h:h + 1]
        recip = jnp.where(d != 0.0, 1.0 / d, 0.0)
        outs.append(num[:, h * OUT_DIM:(h + 1) * OUT_DIM] * recip)
    out_ref[...] = jnp.concatenate(outs, axis=1)


def _divide(num, den16):
    return pl.pallas_call(
        _div_body,
        grid=(NPAD // NB,),
        in_specs=[
            pl.BlockSpec((NB, HEADS * OUT_DIM), lambda i: (i, 0)),
            pl.BlockSpec((NB, 8), lambda i: (i, 0)),
        ],
        out_specs=pl.BlockSpec((NB, HEADS * OUT_DIM), lambda i: (i, 0)),
        out_shape=jax.ShapeDtypeStruct((N, HEADS * OUT_DIM), jnp.float32),
    )(num, den16)


def kernel(x, path_list, W_ih, W_hh, b_ih, b_hh, a):
    idx_tm = path_list.T.reshape(-1).astype(jnp.int32)
    seg = path_list[:, L - 1].astype(jnp.int32)
    seg16 = seg // 16
    m16 = (seg % 16).astype(jnp.float32).reshape(P, 1)
    abig = jnp.repeat(a, OUT_DIM, axis=1)
    irep = jnp.tile(jnp.eye(OUT_DIM, dtype=jnp.float32), (1, HEADS))
    apat = jnp.tile(
        jnp.concatenate([a, jnp.zeros((OUT_DIM, 8 - HEADS), jnp.float32)],
                        axis=1), (1, 16))
    feats = _gather(x, idx_tm)
    payn, payd = _gru_payload(
        feats, W_ih.T, W_hh.T, b_ih.reshape(1, -1), b_hh.reshape(1, -1),
        abig, irep, apat, m16)
    num, den = _scatter(payn, payd, seg, seg16)
    den8 = (den[:, :128] + den[:, 128:]).reshape(NPAD, 8)
    return _divide(num, den8)


# speedup vs baseline: 4.7783x; 1.0350x over previous
import jax
import jax.numpy as jnp
from jax import lax
from jax.experimental import pallas as pl
from jax.experimental.pallas import tpu as pltpu
from jax.experimental.pallas import tpu_sc as plsc

N = 10000
NPAD = 10240
P = 80000
L = 4
IN_DIM = 128
OUT_DIM = 64
HEADS = 4

NC = 2
NS = 16
NW = NC * NS

DROWS = NPAD // 16


def _sc_mesh():
    return plsc.VectorSubcoreMesh(
        core_axis_name="c", subcore_axis_name="s", num_cores=NC, num_subcores=NS)


_SC_PARAMS = pltpu.CompilerParams(use_tc_tiling_on_sc=True)

G_ROWS = P // (NW // L)
G_CHUNK = 400
G_ITERS = G_ROWS // G_CHUNK


def _gather_body(x_hbm, idx_hbm, out_hbm, idx0, idx1, rows0, rows1,
                 semi0, semi1, semg0, semg1):
    wid = lax.axis_index("s") * NC + lax.axis_index("c")
    t = wid % L
    start = (wid // L) * G_ROWS

    def isrc(j):
        return idx_hbm.at[pl.ds(t * P + start + j * G_CHUNK, G_CHUNK)]

    def istart(j, ib, semi):
        @pl.when(j < G_ITERS)
        def _():
            pltpu.async_copy(isrc(j), ib, semi)

    def iwait(j, ib, semi):
        pltpu.make_async_copy(isrc(j), ib, semi).wait()

    def gstart(ib, rb, semg):
        pltpu.async_copy(x_hbm.at[ib], rb, semg)

    def gwait(ib, rb, semg):
        pltpu.make_async_copy(x_hbm.at[ib], rb, semg).wait()

    def store(j, rb):
        pltpu.sync_copy(rb, out_hbm.at[t, pl.ds(start + j * G_CHUNK, G_CHUNK)])

    istart(0, idx0, semi0)
    iwait(0, idx0, semi0)
    gstart(idx0, rows0, semg0)
    istart(1, idx1, semi1)

    def pair(k, _):
        j1 = 2 * k + 1
        iwait(j1, idx1, semi1)
        gstart(idx1, rows1, semg1)
        gwait(idx0, rows0, semg0)
        istart(j1 + 1, idx0, semi0)
        store(j1 - 1, rows0)
        j2 = 2 * k + 2
        iwait(j2, idx0, semi0)
        gstart(idx0, rows0, semg0)
        gwait(idx1, rows1, semg1)
        istart(j2 + 1, idx1, semi1)
        store(j2 - 1, rows1)
        return ()

    lax.fori_loop(0, (G_ITERS - 1) // 2, pair, ())
    gwait(idx0, rows0, semg0)
    store(G_ITERS - 1, rows0)


def _gather(x, idx_tm):
    return pl.kernel(
        _gather_body,
        out_type=jax.ShapeDtypeStruct((L, P, IN_DIM), jnp.float32),
        mesh=_sc_mesh(),
        scratch_types=[
            pltpu.VMEM((G_CHUNK,), jnp.int32),
            pltpu.VMEM((G_CHUNK,), jnp.int32),
            pltpu.VMEM((G_CHUNK, IN_DIM), jnp.float32),
            pltpu.VMEM((G_CHUNK, IN_DIM), jnp.float32),
            pltpu.SemaphoreType.DMA,
            pltpu.SemaphoreType.DMA,
            pltpu.SemaphoreType.DMA,
            pltpu.SemaphoreType.DMA,
        ],
        compiler_params=_SC_PARAMS,
    )(x, idx_tm)


PB = 1000


def _gru_body(f_ref, wi_ref, wh_ref, bi_ref, bh_ref, abig_ref, irep_ref,
              apat_ref, m8_ref, payn_ref, payd_ref):
    def dot(u, v):
        return jnp.dot(u, v, preferred_element_type=jnp.float32)

    wi = wi_ref[...]
    wh = wh_ref[...]
    b = bi_ref[...] + bh_ref[...]
    h = jnp.zeros((PB, OUT_DIM), jnp.float32)
    for t in range(L):
        xt = f_ref[t]
        gh = dot(h, wh)
        s = dot(xt, wi) + gh + b
        rz = jax.nn.sigmoid(s[:, :2 * OUT_DIM])
        r = rz[:, :OUT_DIM]
        z = rz[:, OUT_DIM:]
        n = jnp.tanh(s[:, 2 * OUT_DIM:] + (r - 1.0) * (gh[:, 2 * OUT_DIM:] + bh_ref[:, 2 * OUT_DIM:]))
        h = (1.0 - z) * n + z * h
    payn_ref[...] = jnp.exp(dot(h, abig_ref[...])) * dot(h, irep_ref[...])
    lane = lax.broadcasted_iota(jnp.int32, (PB, 128), 1)
    grp = (lane // 8).astype(jnp.float32)
    mask = ((grp == m8_ref[...]) & (lane % 8 < HEADS)).astype(jnp.float32)
    payd_ref[...] = jnp.exp(dot(h, apat_ref[...])) * mask


def _gru_payload(feats, wi, wh, bi, bh, abig, irep, apat, m8):
    return pl.pallas_call(
        _gru_body,
        grid=(P // PB,),
        in_specs=[
            pl.BlockSpec((L, PB, IN_DIM), lambda i: (0, i, 0)),
            pl.BlockSpec((IN_DIM, 3 * OUT_DIM), lambda i: (0, 0)),
            pl.BlockSpec((OUT_DIM, 3 * OUT_DIM), lambda i: (0, 0)),
            pl.BlockSpec((1, 3 * OUT_DIM), lambda i: (0, 0)),
            pl.BlockSpec((1, 3 * OUT_DIM), lambda i: (0, 0)),
            pl.BlockSpec((OUT_DIM, HEADS * OUT_DIM), lambda i: (0, 0)),
            pl.BlockSpec((OUT_DIM, HEADS * OUT_DIM), lambda i: (0, 0)),
            pl.BlockSpec((OUT_DIM, 128), lambda i: (0, 0)),
            pl.BlockSpec((PB, 1), lambda i: (i, 0)),
        ],
        out_specs=[
            pl.BlockSpec((PB, HEADS * OUT_DIM), lambda i: (i, 0)),
            pl.BlockSpec((PB, 128), lambda i: (i, 0)),
        ],
        out_shape=[
            jax.ShapeDtypeStruct((P, HEADS * OUT_DIM), jnp.float32),
            jax.ShapeDtypeStruct((P, 128), jnp.float32),
        ],
    )(feats, wi, wh, bi, bh, abig, irep, apat, m8)


CH = 160
NCHUNK = P // CH
DCHUNK = P // NC // CH
ZN_CHUNKS = NPAD // CH
ZD_CHUNKS = DROWS // CH


def _scatter_body(payn_hbm, payd_hbm, seg_hbm, seg16_hbm, num_hbm, den_hbm,
                  accn, accd, pbuf0, pbuf1, sbuf0, sbuf1, sem0, sem1):
    c = lax.axis_index("c")
    s = lax.axis_index("s")

    def zrow(i, _):
        for j in range(128 // 16):
            pbuf0[i, pl.ds(j * 16, 16)] = jnp.zeros((16,), jnp.float32)
        return ()

    lax.fori_loop(0, CH, zrow, ())
    for j in range(ZN_CHUNKS // NS):
        pltpu.sync_copy(pbuf0, accn.at[pl.ds((s + j * NS) * CH, CH)])

    @pl.when(s < ZD_CHUNKS)
    def _():
        pltpu.sync_copy(pbuf0, accd.at[pl.ds(s * CH, CH)])
    plsc.subcore_barrier()

    def n_src(j):
        base = (s + j * NS) * CH
        return (seg_hbm.at[pl.ds(base, CH)],
                payn_hbm.at[pl.ds(base, CH), pl.ds(c * 128, 128)])

    def n_start(j, pb, sb, sem):
        @pl.when(s + j * NS < NCHUNK)
        def _():
            sg, pay = n_src(j)
            pltpu.async_copy(sg, sb, sem)
            pltpu.async_copy(pay, pb, sem)

    def n_consume(j, pb, sb, sem):
        @pl.when(s + j * NS < NCHUNK)
        def _():
            sg, pay = n_src(j)
            pltpu.make_async_copy(sg, sb, sem).wait()
            pltpu.make_async_copy(pay, pb, sem).wait()
            pltpu.sync_copy(pb, accn.at[sb], add=True)

    NJ = (NCHUNK + NS - 1) // NS

    n_start(0, pbuf0, sbuf0, sem0)

    def npair(k, _):
        j0 = 2 * k
        n_start(j0 + 1, pbuf1, sbuf1, sem1)
        n_consume(j0, pbuf0, sbuf0, sem0)
        n_start(j0 + 2, pbuf0, sbuf0, sem0)
        n_consume(j0 + 1, pbuf1, sbuf1, sem1)
        return ()

    lax.fori_loop(0, NJ // 2, npair, ())

    def d_src(j):
        base = c * (P // NC) + (s + j * NS) * CH
        return (seg16_hbm.at[pl.ds(base, CH)], payd_hbm.at[pl.ds(base, CH)])

    def d_start(j, pb, sb, sem):
        @pl.when(s + j * NS < DCHUNK)
        def _():
            sg, pay = d_src(j)
            pltpu.async_copy(sg, sb, sem)
            pltpu.async_copy(pay, pb, sem)

    def d_consume(j, pb, sb, sem):
        @pl.when(s + j * NS < DCHUNK)
        def _():
            sg, pay = d_src(j)
            pltpu.make_async_copy(sg, sb, sem).wait()
            pltpu.make_async_copy(pay, pb, sem).wait()
            pltpu.sync_copy(pb, accd.at[sb], add=True)

    NJD = (DCHUNK + NS - 1) // NS

    d_start(0, pbuf0, sbuf0, sem0)

    def dpair(k, _):
        j0 = 2 * k
        d_start(j0 + 1, pbuf1, sbuf1, sem1)
        d_consume(j0, pbuf0, sbuf0, sem0)
        d_start(j0 + 2, pbuf0, sbuf0, sem0)
        d_consume(j0 + 1, pbuf1, sbuf1, sem1)
        return ()

    lax.fori_loop(0, NJD // 2, dpair, ())
    plsc.subcore_barrier()

    for j in range(ZN_CHUNKS // NS):
        base = (s + j * NS) * CH
        pltpu.sync_copy(accn.at[pl.ds(base, CH)],
                        num_hbm.at[pl.ds(base, CH), pl.ds(c * 128, 128)])

    @pl.when(s < ZD_CHUNKS)
    def _():
        pltpu.sync_copy(accd.at[pl.ds(s * CH, CH)],
                        den_hbm.at[pl.ds(s * CH, CH), pl.ds(c * 128, 128)])


def _scatter(payn, payd, seg, seg16):
    return pl.kernel(
        _scatter_body,
        out_type=[
            jax.ShapeDtypeStruct((NPAD, 2 * 128), jnp.float32),
            jax.ShapeDtypeStruct((DROWS, 2 * 128), jnp.float32),
        ],
        mesh=_sc_mesh(),
        scratch_types=[
            pltpu.VMEM_SHARED((NPAD, 128), jnp.float32),
            pltpu.VMEM_SHARED((DROWS, 128), jnp.float32),
            pltpu.VMEM((CH, 128), jnp.float32),
            pltpu.VMEM((CH, 128), jnp.float32),
            pltpu.VMEM((CH,), jnp.int32),
            pltpu.VMEM((CH,), jnp.int32),
            pltpu.SemaphoreType.DMA,
            pltpu.SemaphoreType.DMA,
        ],
        compiler_params=_SC_PARAMS,
    )(payn, payd, seg, seg16)


NB = 2048


def _div_body(num_ref, den_ref, out_ref):
    num = num_ref[...]
    den = den_ref[...]
    outs = []
    for h in range(HEADS):
        d = den[:, h:h + 1]
        recip = jnp.where(d != 0.0, 1.0 / d, 0.0)
        outs.append(num[:, h * OUT_DIM:(h + 1) * OUT_DIM] * recip)
    out_ref[...] = jnp.concatenate(outs, axis=1)


def _divide(num, den16):
    return pl.pallas_call(
        _div_body,
        grid=(NPAD // NB,),
        in_specs=[
            pl.BlockSpec((NB, HEADS * OUT_DIM), lambda i: (i, 0)),
            pl.BlockSpec((NB, 8), lambda i: (i, 0)),
        ],
        out_specs=pl.BlockSpec((NB, HEADS * OUT_DIM), lambda i: (i, 0)),
        out_shape=jax.ShapeDtypeStruct((N, HEADS * OUT_DIM), jnp.float32),
    )(num, den16)


def kernel(x, path_list, W_ih, W_hh, b_ih, b_hh, a):
    idx_tm = path_list.T.reshape(-1).astype(jnp.int32)
    seg = path_list[:, L - 1].astype(jnp.int32)
    seg16 = seg // 16
    m16 = (seg % 16).astype(jnp.float32).reshape(P, 1)
    abig = jnp.repeat(a, OUT_DIM, axis=1)
    irep = jnp.tile(jnp.eye(OUT_DIM, dtype=jnp.float32), (1, HEADS))
    apat = jnp.tile(
        jnp.concatenate([a, jnp.zeros((OUT_DIM, 8 - HEADS), jnp.float32)],
                        axis=1), (1, 16))
    feats = _gather(x, idx_tm)
    payn, payd = _gru_payload(
        feats, W_ih.T, W_hh.T, b_ih.reshape(1, -1), b_hh.reshape(1, -1),
        abig, irep, apat, m16)
    num, den = _scatter(payn, payd, seg, seg16)
    den8 = (den[:, :128] + den[:, 128:]).reshape(NPAD, 8)
    return _divide(num, den8)


# speedup vs baseline: 5.0736x; 1.0618x over previous
import jax
import jax.numpy as jnp
from jax import lax
from jax.experimental import pallas as pl
from jax.experimental.pallas import tpu as pltpu
from jax.experimental.pallas import tpu_sc as plsc

N = 10000
NPAD = 10240
P = 80000
L = 4
IN_DIM = 128
OUT_DIM = 64
HEADS = 4

NC = 2
NS = 16
NW = NC * NS

DROWS = NPAD // 16


def _sc_mesh():
    return plsc.VectorSubcoreMesh(
        core_axis_name="c", subcore_axis_name="s", num_cores=NC, num_subcores=NS)


_SC_PARAMS = pltpu.CompilerParams(use_tc_tiling_on_sc=True)

G_ROWS = P // (NW // L)
G_CHUNK = 400
G_ITERS = G_ROWS // G_CHUNK


def _gather_body(x_hbm, idx_hbm, out_hbm, idx0, idx1, rows0, rows1,
                 semi0, semi1, semg0, semg1):
    wid = lax.axis_index("s") * NC + lax.axis_index("c")
    t = wid % L
    start = (wid // L) * G_ROWS

    def isrc(j):
        return idx_hbm.at[pl.ds(t * P + start + j * G_CHUNK, G_CHUNK)]

    def istart(j, ib, semi):
        @pl.when(j < G_ITERS)
        def _():
            pltpu.async_copy(isrc(j), ib, semi)

    def iwait(j, ib, semi):
        pltpu.make_async_copy(isrc(j), ib, semi).wait()

    def gstart(ib, rb, semg):
        pltpu.async_copy(x_hbm.at[ib], rb, semg)

    def gwait(ib, rb, semg):
        pltpu.make_async_copy(x_hbm.at[ib], rb, semg).wait()

    def store(j, rb):
        pltpu.sync_copy(rb, out_hbm.at[t, pl.ds(start + j * G_CHUNK, G_CHUNK)])

    istart(0, idx0, semi0)
    iwait(0, idx0, semi0)
    gstart(idx0, rows0, semg0)
    istart(1, idx1, semi1)

    def pair(k, _):
        j1 = 2 * k + 1
        iwait(j1, idx1, semi1)
        gstart(idx1, rows1, semg1)
        gwait(idx0, rows0, semg0)
        istart(j1 + 1, idx0, semi0)
        store(j1 - 1, rows0)
        j2 = 2 * k + 2
        iwait(j2, idx0, semi0)
        gstart(idx0, rows0, semg0)
        gwait(idx1, rows1, semg1)
        istart(j2 + 1, idx1, semi1)
        store(j2 - 1, rows1)
        return ()

    lax.fori_loop(0, (G_ITERS - 1) // 2, pair, ())
    gwait(idx0, rows0, semg0)
    store(G_ITERS - 1, rows0)


def _gather(x, idx_tm):
    return pl.kernel(
        _gather_body,
        out_type=jax.ShapeDtypeStruct((L, P, IN_DIM), jnp.float32),
        mesh=_sc_mesh(),
        scratch_types=[
            pltpu.VMEM((G_CHUNK,), jnp.int32),
            pltpu.VMEM((G_CHUNK,), jnp.int32),
            pltpu.VMEM((G_CHUNK, IN_DIM), jnp.float32),
            pltpu.VMEM((G_CHUNK, IN_DIM), jnp.float32),
            pltpu.SemaphoreType.DMA,
            pltpu.SemaphoreType.DMA,
            pltpu.SemaphoreType.DMA,
            pltpu.SemaphoreType.DMA,
        ],
        compiler_params=_SC_PARAMS,
    )(x, idx_tm)


PB = 2000


def _gru_body(f_ref, wi_ref, wh_ref, bi_ref, bh_ref, abig_ref, irep_ref,
              apat_ref, m8_ref, payn_ref, payd_ref):
    def dot(u, v):
        return jnp.dot(u, v, preferred_element_type=jnp.float32)

    wi = wi_ref[...]
    wh = wh_ref[...]
    b = bi_ref[...] + bh_ref[...]
    h = jnp.zeros((PB, OUT_DIM), jnp.float32)
    for t in range(L):
        xt = f_ref[t]
        gh = dot(h, wh)
        s = dot(xt, wi) + gh + b
        rz = jax.nn.sigmoid(s[:, :2 * OUT_DIM])
        r = rz[:, :OUT_DIM]
        z = rz[:, OUT_DIM:]
        n = jnp.tanh(s[:, 2 * OUT_DIM:] + (r - 1.0) * (gh[:, 2 * OUT_DIM:] + bh_ref[:, 2 * OUT_DIM:]))
        h = (1.0 - z) * n + z * h
    payn_ref[...] = jnp.exp(dot(h, abig_ref[...])) * dot(h, irep_ref[...])
    lane = lax.broadcasted_iota(jnp.int32, (PB, 128), 1)
    grp = (lane // 8).astype(jnp.float32)
    mask = ((grp == m8_ref[...]) & (lane % 8 < HEADS)).astype(jnp.float32)
    payd_ref[...] = jnp.exp(dot(h, apat_ref[...])) * mask


def _gru_payload(feats, wi, wh, bi, bh, abig, irep, apat, m8):
    return pl.pallas_call(
        _gru_body,
        grid=(P // PB,),
        in_specs=[
            pl.BlockSpec((L, PB, IN_DIM), lambda i: (0, i, 0)),
            pl.BlockSpec((IN_DIM, 3 * OUT_DIM), lambda i: (0, 0)),
            pl.BlockSpec((OUT_DIM, 3 * OUT_DIM), lambda i: (0, 0)),
            pl.BlockSpec((1, 3 * OUT_DIM), lambda i: (0, 0)),
            pl.BlockSpec((1, 3 * OUT_DIM), lambda i: (0, 0)),
            pl.BlockSpec((OUT_DIM, HEADS * OUT_DIM), lambda i: (0, 0)),
            pl.BlockSpec((OUT_DIM, HEADS * OUT_DIM), lambda i: (0, 0)),
            pl.BlockSpec((OUT_DIM, 128), lambda i: (0, 0)),
            pl.BlockSpec((PB, 1), lambda i: (i, 0)),
        ],
        out_specs=[
            pl.BlockSpec((PB, HEADS * OUT_DIM), lambda i: (i, 0)),
            pl.BlockSpec((PB, 128), lambda i: (i, 0)),
        ],
        out_shape=[
            jax.ShapeDtypeStruct((P, HEADS * OUT_DIM), jnp.float32),
            jax.ShapeDtypeStruct((P, 128), jnp.float32),
        ],
    )(feats, wi, wh, bi, bh, abig, irep, apat, m8)


CH = 160
NCHUNK = P // CH
DCHUNK = P // NC // CH
ZN_CHUNKS = NPAD // CH
ZD_CHUNKS = DROWS // CH


def _scatter_body(payn_hbm, payd_hbm, seg_hbm, seg16_hbm, num_hbm, den_hbm,
                  accn, accd, pbuf0, pbuf1, sbuf0, sbuf1, sem0, sem1):
    c = lax.axis_index("c")
    s = lax.axis_index("s")

    def zrow(i, _):
        for j in range(128 // 16):
            pbuf0[i, pl.ds(j * 16, 16)] = jnp.zeros((16,), jnp.float32)
        return ()

    lax.fori_loop(0, CH, zrow, ())
    for j in range(ZN_CHUNKS // NS):
        pltpu.sync_copy(pbuf0, accn.at[pl.ds((s + j * NS) * CH, CH)])

    @pl.when(s < ZD_CHUNKS)
    def _():
        pltpu.sync_copy(pbuf0, accd.at[pl.ds(s * CH, CH)])
    plsc.subcore_barrier()

    def n_src(j):
        base = (s + j * NS) * CH
        return (seg_hbm.at[pl.ds(base, CH)],
                payn_hbm.at[pl.ds(base, CH), pl.ds(c * 128, 128)])

    def n_start(j, pb, sb, sem):
        @pl.when(s + j * NS < NCHUNK)
        def _():
            sg, pay = n_src(j)
            pltpu.async_copy(sg, sb, sem)
            pltpu.async_copy(pay, pb, sem)

    def n_consume(j, pb, sb, sem):
        @pl.when(s + j * NS < NCHUNK)
        def _():
            sg, pay = n_src(j)
            pltpu.make_async_copy(sg, sb, sem).wait()
            pltpu.make_async_copy(pay, pb, sem).wait()
            pltpu.sync_copy(pb, accn.at[sb], add=True)

    NJ = (NCHUNK + NS - 1) // NS

    n_start(0, pbuf0, sbuf0, sem0)

    def npair(k, _):
        j0 = 2 * k
        n_start(j0 + 1, pbuf1, sbuf1, sem1)
        n_consume(j0, pbuf0, sbuf0, sem0)
        n_start(j0 + 2, pbuf0, sbuf0, sem0)
        n_consume(j0 + 1, pbuf1, sbuf1, sem1)
        return ()

    lax.fori_loop(0, NJ // 2, npair, ())

    def d_src(j):
        base = c * (P // NC) + (s + j * NS) * CH
        return (seg16_hbm.at[pl.ds(base, CH)], payd_hbm.at[pl.ds(base, CH)])

    def d_start(j, pb, sb, sem):
        @pl.when(s + j * NS < DCHUNK)
        def _():
            sg, pay = d_src(j)
            pltpu.async_copy(sg, sb, sem)
            pltpu.async_copy(pay, pb, sem)

    def d_consume(j, pb, sb, sem):
        @pl.when(s + j * NS < DCHUNK)
        def _():
            sg, pay = d_src(j)
            pltpu.make_async_copy(sg, sb, sem).wait()
            pltpu.make_async_copy(pay, pb, sem).wait()
            pltpu.sync_copy(pb, accd.at[sb], add=True)

    NJD = (DCHUNK + NS - 1) // NS

    d_start(0, pbuf0, sbuf0, sem0)

    def dpair(k, _):
        j0 = 2 * k
        d_start(j0 + 1, pbuf1, sbuf1, sem1)
        d_consume(j0, pbuf0, sbuf0, sem0)
        d_start(j0 + 2, pbuf0, sbuf0, sem0)
        d_consume(j0 + 1, pbuf1, sbuf1, sem1)
        return ()

    lax.fori_loop(0, NJD // 2, dpair, ())
    plsc.subcore_barrier()

    for j in range(ZN_CHUNKS // NS):
        base = (s + j * NS) * CH
        pltpu.sync_copy(accn.at[pl.ds(base, CH)],
                        num_hbm.at[pl.ds(base, CH), pl.ds(c * 128, 128)])

    @pl.when(s < ZD_CHUNKS)
    def _():
        pltpu.sync_copy(accd.at[pl.ds(s * CH, CH)],
                        den_hbm.at[pl.ds(s * CH, CH), pl.ds(c * 128, 128)])


def _scatter(payn, payd, seg, seg16):
    return pl.kernel(
        _scatter_body,
        out_type=[
            jax.ShapeDtypeStruct((NPAD, 2 * 128), jnp.float32),
            jax.ShapeDtypeStruct((DROWS, 2 * 128), jnp.float32),
        ],
        mesh=_sc_mesh(),
        scratch_types=[
            pltpu.VMEM_SHARED((NPAD, 128), jnp.float32),
            pltpu.VMEM_SHARED((DROWS, 128), jnp.float32),
            pltpu.VMEM((CH, 128), jnp.float32),
            pltpu.VMEM((CH, 128), jnp.float32),
            pltpu.VMEM((CH,), jnp.int32),
            pltpu.VMEM((CH,), jnp.int32),
            pltpu.SemaphoreType.DMA,
            pltpu.SemaphoreType.DMA,
        ],
        compiler_params=_SC_PARAMS,
    )(payn, payd, seg, seg16)


NB = 2048


def _div_body(num_ref, den_ref, out_ref):
    num = num_ref[...]
    den = den_ref[...]
    outs = []
    for h in range(HEADS):
        d = den[:, h:h + 1]
        recip = jnp.where(d != 0.0, 1.0 / d, 0.0)
        outs.append(num[:, h * OUT_DIM:(h + 1) * OUT_DIM] * recip)
    out_ref[...] = jnp.concatenate(outs, axis=1)


def _divide(num, den16):
    return pl.pallas_call(
        _div_body,
        grid=(NPAD // NB,),
        in_specs=[
            pl.BlockSpec((NB, HEADS * OUT_DIM), lambda i: (i, 0)),
            pl.BlockSpec((NB, 8), lambda i: (i, 0)),
        ],
        out_specs=pl.BlockSpec((NB, HEADS * OUT_DIM), lambda i: (i, 0)),
        out_shape=jax.ShapeDtypeStruct((N, HEADS * OUT_DIM), jnp.float32),
    )(num, den16)


def kernel(x, path_list, W_ih, W_hh, b_ih, b_hh, a):
    idx_tm = path_list.T.reshape(-1).astype(jnp.int32)
    seg = path_list[:, L - 1].astype(jnp.int32)
    seg16 = seg // 16
    m16 = (seg % 16).astype(jnp.float32).reshape(P, 1)
    abig = jnp.repeat(a, OUT_DIM, axis=1)
    irep = jnp.tile(jnp.eye(OUT_DIM, dtype=jnp.float32), (1, HEADS))
    apat = jnp.tile(
        jnp.concatenate([a, jnp.zeros((OUT_DIM, 8 - HEADS), jnp.float32)],
                        axis=1), (1, 16))
    feats = _gather(x, idx_tm)
    payn, payd = _gru_payload(
        feats, W_ih.T, W_hh.T, b_ih.reshape(1, -1), b_hh.reshape(1, -1),
        abig, irep, apat, m16)
    num, den = _scatter(payn, payd, seg, seg16)
    den8 = (den[:, :128] + den[:, 128:]).reshape(NPAD, 8)
    return _divide(num, den8)


# speedup vs baseline: 5.0927x; 1.0038x over previous
import jax
import jax.numpy as jnp
from jax import lax
from jax.experimental import pallas as pl
from jax.experimental.pallas import tpu as pltpu
from jax.experimental.pallas import tpu_sc as plsc

N = 10000
NPAD = 10240
P = 80000
L = 4
IN_DIM = 128
OUT_DIM = 64
HEADS = 4

NC = 2
NS = 16
NW = NC * NS

DROWS = NPAD // 16


def _sc_mesh():
    return plsc.VectorSubcoreMesh(
        core_axis_name="c", subcore_axis_name="s", num_cores=NC, num_subcores=NS)


_SC_PARAMS = pltpu.CompilerParams(use_tc_tiling_on_sc=True)

G_ROWS = P // (NW // L)
G_CHUNK = 400
G_ITERS = G_ROWS // G_CHUNK


def _gather_body(x_hbm, idx_hbm, out_hbm, idx0, idx1, rows0, rows1,
                 semi0, semi1, semg0, semg1):
    wid = lax.axis_index("s") * NC + lax.axis_index("c")
    t = wid % L
    start = (wid // L) * G_ROWS

    def isrc(j):
        return idx_hbm.at[pl.ds(t * P + start + j * G_CHUNK, G_CHUNK)]

    def istart(j, ib, semi):
        @pl.when(j < G_ITERS)
        def _():
            pltpu.async_copy(isrc(j), ib, semi)

    def iwait(j, ib, semi):
        pltpu.make_async_copy(isrc(j), ib, semi).wait()

    def gstart(ib, rb, semg):
        pltpu.async_copy(x_hbm.at[ib], rb, semg)

    def gwait(ib, rb, semg):
        pltpu.make_async_copy(x_hbm.at[ib], rb, semg).wait()

    def store(j, rb):
        pltpu.sync_copy(rb, out_hbm.at[t, pl.ds(start + j * G_CHUNK, G_CHUNK)])

    istart(0, idx0, semi0)
    iwait(0, idx0, semi0)
    gstart(idx0, rows0, semg0)
    istart(1, idx1, semi1)

    def pair(k, _):
        j1 = 2 * k + 1
        iwait(j1, idx1, semi1)
        gstart(idx1, rows1, semg1)
        gwait(idx0, rows0, semg0)
        istart(j1 + 1, idx0, semi0)
        store(j1 - 1, rows0)
        j2 = 2 * k + 2
        iwait(j2, idx0, semi0)
        gstart(idx0, rows0, semg0)
        gwait(idx1, rows1, semg1)
        istart(j2 + 1, idx1, semi1)
        store(j2 - 1, rows1)
        return ()

    lax.fori_loop(0, (G_ITERS - 1) // 2, pair, ())
    gwait(idx0, rows0, semg0)
    store(G_ITERS - 1, rows0)


def _gather(x, idx_tm):
    return pl.kernel(
        _gather_body,
        out_type=jax.ShapeDtypeStruct((L, P, IN_DIM), jnp.float32),
        mesh=_sc_mesh(),
        scratch_types=[
            pltpu.VMEM((G_CHUNK,), jnp.int32),
            pltpu.VMEM((G_CHUNK,), jnp.int32),
            pltpu.VMEM((G_CHUNK, IN_DIM), jnp.float32),
            pltpu.VMEM((G_CHUNK, IN_DIM), jnp.float32),
            pltpu.SemaphoreType.DMA,
            pltpu.SemaphoreType.DMA,
            pltpu.SemaphoreType.DMA,
            pltpu.SemaphoreType.DMA,
        ],
        compiler_params=_SC_PARAMS,
    )(x, idx_tm)


PB = 4000


def _gru_body(f_ref, wi_ref, wh_ref, bi_ref, bh_ref, abig_ref, irep_ref,
              apat_ref, m8_ref, payn_ref, payd_ref):
    def dot(u, v):
        return jnp.dot(u, v, preferred_element_type=jnp.float32)

    wi = wi_ref[...]
    wh = wh_ref[...]
    b = bi_ref[...] + bh_ref[...]
    h = jnp.zeros((PB, OUT_DIM), jnp.float32)
    for t in range(L):
        xt = f_ref[t]
        gh = dot(h, wh)
        s = dot(xt, wi) + gh + b
        rz = jax.nn.sigmoid(s[:, :2 * OUT_DIM])
        r = rz[:, :OUT_DIM]
        z = rz[:, OUT_DIM:]
        n = jnp.tanh(s[:, 2 * OUT_DIM:] + (r - 1.0) * (gh[:, 2 * OUT_DIM:] + bh_ref[:, 2 * OUT_DIM:]))
        h = (1.0 - z) * n + z * h
    payn_ref[...] = jnp.exp(dot(h, abig_ref[...])) * dot(h, irep_ref[...])
    lane = lax.broadcasted_iota(jnp.int32, (PB, 128), 1)
    grp = (lane // 8).astype(jnp.float32)
    mask = ((grp == m8_ref[...]) & (lane % 8 < HEADS)).astype(jnp.float32)
    payd_ref[...] = jnp.exp(dot(h, apat_ref[...])) * mask


def _gru_payload(feats, wi, wh, bi, bh, abig, irep, apat, m8):
    return pl.pallas_call(
        _gru_body,
        grid=(P // PB,),
        in_specs=[
            pl.BlockSpec((L, PB, IN_DIM), lambda i: (0, i, 0)),
            pl.BlockSpec((IN_DIM, 3 * OUT_DIM), lambda i: (0, 0)),
            pl.BlockSpec((OUT_DIM, 3 * OUT_DIM), lambda i: (0, 0)),
            pl.BlockSpec((1, 3 * OUT_DIM), lambda i: (0, 0)),
            pl.BlockSpec((1, 3 * OUT_DIM), lambda i: (0, 0)),
            pl.BlockSpec((OUT_DIM, HEADS * OUT_DIM), lambda i: (0, 0)),
            pl.BlockSpec((OUT_DIM, HEADS * OUT_DIM), lambda i: (0, 0)),
            pl.BlockSpec((OUT_DIM, 128), lambda i: (0, 0)),
            pl.BlockSpec((PB, 1), lambda i: (i, 0)),
        ],
        out_specs=[
            pl.BlockSpec((PB, HEADS * OUT_DIM), lambda i: (i, 0)),
            pl.BlockSpec((PB, 128), lambda i: (i, 0)),
        ],
        out_shape=[
            jax.ShapeDtypeStruct((P, HEADS * OUT_DIM), jnp.float32),
            jax.ShapeDtypeStruct((P, 128), jnp.float32),
        ],
    )(feats, wi, wh, bi, bh, abig, irep, apat, m8)


CH = 160
NCHUNK = P // CH
DCHUNK = P // NC // CH
ZN_CHUNKS = NPAD // CH
ZD_CHUNKS = DROWS // CH


def _scatter_body(payn_hbm, payd_hbm, seg_hbm, seg16_hbm, num_hbm, den_hbm,
                  accn, accd, pbuf0, pbuf1, sbuf0, sbuf1, sem0, sem1):
    c = lax.axis_index("c")
    s = lax.axis_index("s")

    def zrow(i, _):
        for j in range(128 // 16):
            pbuf0[i, pl.ds(j * 16, 16)] = jnp.zeros((16,), jnp.float32)
        return ()

    lax.fori_loop(0, CH, zrow, ())
    for j in range(ZN_CHUNKS // NS):
        pltpu.sync_copy(pbuf0, accn.at[pl.ds((s + j * NS) * CH, CH)])

    @pl.when(s < ZD_CHUNKS)
    def _():
        pltpu.sync_copy(pbuf0, accd.at[pl.ds(s * CH, CH)])
    plsc.subcore_barrier()

    def n_src(j):
        base = (s + j * NS) * CH
        return (seg_hbm.at[pl.ds(base, CH)],
                payn_hbm.at[pl.ds(base, CH), pl.ds(c * 128, 128)])

    def n_start(j, pb, sb, sem):
        @pl.when(s + j * NS < NCHUNK)
        def _():
            sg, pay = n_src(j)
            pltpu.async_copy(sg, sb, sem)
            pltpu.async_copy(pay, pb, sem)

    def n_consume(j, pb, sb, sem):
        @pl.when(s + j * NS < NCHUNK)
        def _():
            sg, pay = n_src(j)
            pltpu.make_async_copy(sg, sb, sem).wait()
            pltpu.make_async_copy(pay, pb, sem).wait()
            pltpu.sync_copy(pb, accn.at[sb], add=True)

    NJ = (NCHUNK + NS - 1) // NS

    n_start(0, pbuf0, sbuf0, sem0)

    def npair(k, _):
        j0 = 2 * k
        n_start(j0 + 1, pbuf1, sbuf1, sem1)
        n_consume(j0, pbuf0, sbuf0, sem0)
        n_start(j0 + 2, pbuf0, sbuf0, sem0)
        n_consume(j0 + 1, pbuf1, sbuf1, sem1)
        return ()

    lax.fori_loop(0, NJ // 2, npair, ())

    def d_src(j):
        base = c * (P // NC) + (s + j * NS) * CH
        return (seg16_hbm.at[pl.ds(base, CH)], payd_hbm.at[pl.ds(base, CH)])

    def d_start(j, pb, sb, sem):
        @pl.when(s + j * NS < DCHUNK)
        def _():
            sg, pay = d_src(j)
            pltpu.async_copy(sg, sb, sem)
            pltpu.async_copy(pay, pb, sem)

    def d_consume(j, pb, sb, sem):
        @pl.when(s + j * NS < DCHUNK)
        def _():
            sg, pay = d_src(j)
            pltpu.make_async_copy(sg, sb, sem).wait()
            pltpu.make_async_copy(pay, pb, sem).wait()
            pltpu.sync_copy(pb, accd.at[sb], add=True)

    NJD = (DCHUNK + NS - 1) // NS

    d_start(0, pbuf0, sbuf0, sem0)

    def dpair(k, _):
        j0 = 2 * k
        d_start(j0 + 1, pbuf1, sbuf1, sem1)
        d_consume(j0, pbuf0, sbuf0, sem0)
        d_start(j0 + 2, pbuf0, sbuf0, sem0)
        d_consume(j0 + 1, pbuf1, sbuf1, sem1)
        return ()

    lax.fori_loop(0, NJD // 2, dpair, ())
    plsc.subcore_barrier()

    for j in range(ZN_CHUNKS // NS):
        base = (s + j * NS) * CH
        pltpu.sync_copy(accn.at[pl.ds(base, CH)],
                        num_hbm.at[pl.ds(base, CH), pl.ds(c * 128, 128)])

    @pl.when(s < ZD_CHUNKS)
    def _():
        pltpu.sync_copy(accd.at[pl.ds(s * CH, CH)],
                        den_hbm.at[pl.ds(s * CH, CH), pl.ds(c * 128, 128)])


def _scatter(payn, payd, seg, seg16):
    return pl.kernel(
        _scatter_body,
        out_type=[
            jax.ShapeDtypeStruct((NPAD, 2 * 128), jnp.float32),
            jax.ShapeDtypeStruct((DROWS, 2 * 128), jnp.float32),
        ],
        mesh=_sc_mesh(),
        scratch_types=[
            pltpu.VMEM_SHARED((NPAD, 128), jnp.float32),
            pltpu.VMEM_SHARED((DROWS, 128), jnp.float32),
            pltpu.VMEM((CH, 128), jnp.float32),
            pltpu.VMEM((CH, 128), jnp.float32),
            pltpu.VMEM((CH,), jnp.int32),
            pltpu.VMEM((CH,), jnp.int32),
            pltpu.SemaphoreType.DMA,
            pltpu.SemaphoreType.DMA,
        ],
        compiler_params=_SC_PARAMS,
    )(payn, payd, seg, seg16)


NB = 2048


def _div_body(num_ref, den_ref, out_ref):
    num = num_ref[...]
    den = den_ref[...]
    outs = []
    for h in range(HEADS):
        d = den[:, h:h + 1]
        recip = jnp.where(d != 0.0, 1.0 / d, 0.0)
        outs.append(num[:, h * OUT_DIM:(h + 1) * OUT_DIM] * recip)
    out_ref[...] = jnp.concatenate(outs, axis=1)


def _divide(num, den16):
    return pl.pallas_call(
        _div_body,
        grid=(NPAD // NB,),
        in_specs=[
            pl.BlockSpec((NB, HEADS * OUT_DIM), lambda i: (i, 0)),
            pl.BlockSpec((NB, 8), lambda i: (i, 0)),
        ],
        out_specs=pl.BlockSpec((NB, HEADS * OUT_DIM), lambda i: (i, 0)),
        out_shape=jax.ShapeDtypeStruct((N, HEADS * OUT_DIM), jnp.float32),
    )(num, den16)


def kernel(x, path_list, W_ih, W_hh, b_ih, b_hh, a):
    idx_tm = path_list.T.reshape(-1).astype(jnp.int32)
    seg = path_list[:, L - 1].astype(jnp.int32)
    seg16 = seg // 16
    m16 = (seg % 16).astype(jnp.float32).reshape(P, 1)
    abig = jnp.repeat(a, OUT_DIM, axis=1)
    irep = jnp.tile(jnp.eye(OUT_DIM, dtype=jnp.float32), (1, HEADS))
    apat = jnp.tile(
        jnp.concatenate([a, jnp.zeros((OUT_DIM, 8 - HEADS), jnp.float32)],
                        axis=1), (1, 16))
    feats = _gather(x, idx_tm)
    payn, payd = _gru_payload(
        feats, W_ih.T, W_hh.T, b_ih.reshape(1, -1), b_hh.reshape(1, -1),
        abig, irep, apat, m16)
    num, den = _scatter(payn, payd, seg, seg16)
    den8 = (den[:, :128] + den[:, 128:]).reshape(NPAD, 8)
    return _divide(num, den8)


# speedup vs baseline: 5.8856x; 1.1557x over previous
import jax
import jax.numpy as jnp
from jax import lax
from jax.experimental import pallas as pl
from jax.experimental.pallas import tpu as pltpu
from jax.experimental.pallas import tpu_sc as plsc

N = 10000
NPAD = 10240
P = 80000
L = 4
IN_DIM = 128
OUT_DIM = 64
HEADS = 4

NC = 2
NS = 16
NW = NC * NS

DROWS = NPAD // 16


def _sc_mesh():
    return plsc.VectorSubcoreMesh(
        core_axis_name="c", subcore_axis_name="s", num_cores=NC, num_subcores=NS)


_SC_PARAMS = pltpu.CompilerParams(use_tc_tiling_on_sc=True)

G_ROWS = P // (NW // L)
G_CHUNK = 192
G_ITERS = 52
G_TAIL = G_ROWS - G_ITERS * G_CHUNK
XL_CHUNK = 200
XL_CHUNKS = N // XL_CHUNK


def _gather_body(x_hbm, idx_hbm, out_hbm, xs, idx0, idx1, rows0, rows1,
                 semi0, semi1, semg0, semg1):
    c = lax.axis_index("c")
    s = lax.axis_index("s")
    wid = s * NC + c
    t = wid % L
    start = (wid // L) * G_ROWS

    for j in range((XL_CHUNKS + NS - 1) // NS):
        chunk = s + j * NS

        @pl.when(chunk < XL_CHUNKS)
        def _():
            pltpu.sync_copy(x_hbm.at[pl.ds(chunk * XL_CHUNK, XL_CHUNK)],
                            xs.at[pl.ds(chunk * XL_CHUNK, XL_CHUNK)])
    plsc.subcore_barrier()

    def isrc(j):
        return idx_hbm.at[pl.ds(t * P + start + j * G_CHUNK, G_CHUNK)]

    def istart(j, ib, semi):
        @pl.when(j < G_ITERS)
        def _():
            pltpu.async_copy(isrc(j), ib, semi)

    def iwait(j, ib, semi):
        pltpu.make_async_copy(isrc(j), ib, semi).wait()

    def gstart(ib, rb, semg):
        pltpu.async_copy(xs.at[ib], rb, semg)

    def gwait(ib, rb, semg):
        pltpu.make_async_copy(xs.at[ib], rb, semg).wait()

    def store(j, rb):
        pltpu.sync_copy(rb, out_hbm.at[t, pl.ds(start + j * G_CHUNK, G_CHUNK)])

    istart(0, idx0, semi0)
    iwait(0, idx0, semi0)
    gstart(idx0, rows0, semg0)
    istart(1, idx1, semi1)

    def pair(k, _):
        j1 = 2 * k + 1
        iwait(j1, idx1, semi1)
        gstart(idx1, rows1, semg1)
        gwait(idx0, rows0, semg0)
        istart(j1 + 1, idx0, semi0)
        store(j1 - 1, rows0)
        j2 = 2 * k + 2
        iwait(j2, idx0, semi0)
        gstart(idx0, rows0, semg0)
        gwait(idx1, rows1, semg1)
        istart(j2 + 1, idx1, semi1)
        store(j2 - 1, rows1)
        return ()

    lax.fori_loop(0, (G_ITERS - 2) // 2, pair, ())
    j1 = G_ITERS - 1
    iwait(j1, idx1, semi1)
    gstart(idx1, rows1, semg1)
    gwait(idx0, rows0, semg0)
    store(j1 - 1, rows0)
    gwait(idx1, rows1, semg1)
    store(j1, rows1)
    tb = start + G_ITERS * G_CHUNK
    pltpu.sync_copy(idx_hbm.at[pl.ds(t * P + tb, G_TAIL)], idx0.at[pl.ds(0, G_TAIL)])
    pltpu.async_copy(xs.at[idx0.at[pl.ds(0, G_TAIL)]], rows0.at[pl.ds(0, G_TAIL)], semg0).wait()
    pltpu.sync_copy(rows0.at[pl.ds(0, G_TAIL)], out_hbm.at[t, pl.ds(tb, G_TAIL)])


def _gather(x, idx_tm):
    return pl.kernel(
        _gather_body,
        out_type=jax.ShapeDtypeStruct((L, P, IN_DIM), jnp.float32),
        mesh=_sc_mesh(),
        scratch_types=[
            pltpu.VMEM_SHARED((N, IN_DIM), jnp.float32),
            pltpu.VMEM((G_CHUNK,), jnp.int32),
            pltpu.VMEM((G_CHUNK,), jnp.int32),
            pltpu.VMEM((G_CHUNK, IN_DIM), jnp.float32),
            pltpu.VMEM((G_CHUNK, IN_DIM), jnp.float32),
            pltpu.SemaphoreType.DMA,
            pltpu.SemaphoreType.DMA,
            pltpu.SemaphoreType.DMA,
            pltpu.SemaphoreType.DMA,
        ],
        compiler_params=_SC_PARAMS,
    )(x, idx_tm)


PB = 4000


def _gru_body(f_ref, wi_ref, wh_ref, bi_ref, bh_ref, abig_ref, irep_ref,
              apat_ref, m8_ref, payn_ref, payd_ref):
    def dot(u, v):
        return jnp.dot(u, v, preferred_element_type=jnp.float32)

    wi = wi_ref[...]
    wh = wh_ref[...]
    b = bi_ref[...] + bh_ref[...]
    h = jnp.zeros((PB, OUT_DIM), jnp.float32)
    for t in range(L):
        xt = f_ref[t]
        gh = dot(h, wh)
        s = dot(xt, wi) + gh + b
        rz = jax.nn.sigmoid(s[:, :2 * OUT_DIM])
        r = rz[:, :OUT_DIM]
        z = rz[:, OUT_DIM:]
        n = jnp.tanh(s[:, 2 * OUT_DIM:] + (r - 1.0) * (gh[:, 2 * OUT_DIM:] + bh_ref[:, 2 * OUT_DIM:]))
        h = (1.0 - z) * n + z * h
    payn_ref[...] = jnp.exp(dot(h, abig_ref[...])) * dot(h, irep_ref[...])
    lane = lax.broadcasted_iota(jnp.int32, (PB, 128), 1)
    grp = (lane // 8).astype(jnp.float32)
    mask = ((grp == m8_ref[...]) & (lane % 8 < HEADS)).astype(jnp.float32)
    payd_ref[...] = jnp.exp(dot(h, apat_ref[...])) * mask


def _gru_payload(feats, wi, wh, bi, bh, abig, irep, apat, m8):
    return pl.pallas_call(
        _gru_body,
        grid=(P // PB,),
        in_specs=[
            pl.BlockSpec((L, PB, IN_DIM), lambda i: (0, i, 0)),
            pl.BlockSpec((IN_DIM, 3 * OUT_DIM), lambda i: (0, 0)),
            pl.BlockSpec((OUT_DIM, 3 * OUT_DIM), lambda i: (0, 0)),
            pl.BlockSpec((1, 3 * OUT_DIM), lambda i: (0, 0)),
            pl.BlockSpec((1, 3 * OUT_DIM), lambda i: (0, 0)),
            pl.BlockSpec((OUT_DIM, HEADS * OUT_DIM), lambda i: (0, 0)),
            pl.BlockSpec((OUT_DIM, HEADS * OUT_DIM), lambda i: (0, 0)),
            pl.BlockSpec((OUT_DIM, 128), lambda i: (0, 0)),
            pl.BlockSpec((PB, 1), lambda i: (i, 0)),
        ],
        out_specs=[
            pl.BlockSpec((PB, HEADS * OUT_DIM), lambda i: (i, 0)),
            pl.BlockSpec((PB, 128), lambda i: (i, 0)),
        ],
        out_shape=[
            jax.ShapeDtypeStruct((P, HEADS * OUT_DIM), jnp.float32),
            jax.ShapeDtypeStruct((P, 128), jnp.float32),
        ],
    )(feats, wi, wh, bi, bh, abig, irep, apat, m8)


CH = 160
NCHUNK = P // CH
DCHUNK = P // NC // CH
ZN_CHUNKS = NPAD // CH
ZD_CHUNKS = DROWS // CH


def _scatter_body(payn_hbm, payd_hbm, seg_hbm, seg16_hbm, num_hbm, den_hbm,
                  accn, accd, pbuf0, pbuf1, sbuf0, sbuf1, sem0, sem1):
    c = lax.axis_index("c")
    s = lax.axis_index("s")

    def zrow(i, _):
        for j in range(128 // 16):
            pbuf0[i, pl.ds(j * 16, 16)] = jnp.zeros((16,), jnp.float32)
        return ()

    lax.fori_loop(0, CH, zrow, ())
    for j in range(ZN_CHUNKS // NS):
        pltpu.sync_copy(pbuf0, accn.at[pl.ds((s + j * NS) * CH, CH)])

    @pl.when(s < ZD_CHUNKS)
    def _():
        pltpu.sync_copy(pbuf0, accd.at[pl.ds(s * CH, CH)])
    plsc.subcore_barrier()

    def n_src(j):
        base = (s + j * NS) * CH
        return (seg_hbm.at[pl.ds(base, CH)],
                payn_hbm.at[pl.ds(base, CH), pl.ds(c * 128, 128)])

    def n_start(j, pb, sb, sem):
        @pl.when(s + j * NS < NCHUNK)
        def _():
            sg, pay = n_src(j)
            pltpu.async_copy(sg, sb, sem)
            pltpu.async_copy(pay, pb, sem)

    def n_consume(j, pb, sb, sem):
        @pl.when(s + j * NS < NCHUNK)
        def _():
            sg, pay = n_src(j)
            pltpu.make_async_copy(sg, sb, sem).wait()
            pltpu.make_async_copy(pay, pb, sem).wait()
            pltpu.sync_copy(pb, accn.at[sb], add=True)

    NJ = (NCHUNK + NS - 1) // NS

    n_start(0, pbuf0, sbuf0, sem0)

    def npair(k, _):
        j0 = 2 * k
        n_start(j0 + 1, pbuf1, sbuf1, sem1)
        n_consume(j0, pbuf0, sbuf0, sem0)
        n_start(j0 + 2, pbuf0, sbuf0, sem0)
        n_consume(j0 + 1, pbuf1, sbuf1, sem1)
        return ()

    lax.fori_loop(0, NJ // 2, npair, ())

    def d_src(j):
        base = c * (P // NC) + (s + j * NS) * CH
        return (seg16_hbm.at[pl.ds(base, CH)], payd_hbm.at[pl.ds(base, CH)])

    def d_start(j, pb, sb, sem):
        @pl.when(s + j * NS < DCHUNK)
        def _():
            sg, pay = d_src(j)
            pltpu.async_copy(sg, sb, sem)
            pltpu.async_copy(pay, pb, sem)

    def d_consume(j, pb, sb, sem):
        @pl.when(s + j * NS < DCHUNK)
        def _():
            sg, pay = d_src(j)
            pltpu.make_async_copy(sg, sb, sem).wait()
            pltpu.make_async_copy(pay, pb, sem).wait()
            pltpu.sync_copy(pb, accd.at[sb], add=True)

    NJD = (DCHUNK + NS - 1) // NS

    d_start(0, pbuf0, sbuf0, sem0)

    def dpair(k, _):
        j0 = 2 * k
        d_start(j0 + 1, pbuf1, sbuf1, sem1)
        d_consume(j0, pbuf0, sbuf0, sem0)
        d_start(j0 + 2, pbuf0, sbuf0, sem0)
        d_consume(j0 + 1, pbuf1, sbuf1, sem1)
        return ()

    lax.fori_loop(0, NJD // 2, dpair, ())
    plsc.subcore_barrier()

    for j in range(ZN_CHUNKS // NS):
        base = (s + j * NS) * CH
        pltpu.sync_copy(accn.at[pl.ds(base, CH)],
                        num_hbm.at[pl.ds(base, CH), pl.ds(c * 128, 128)])

    @pl.when(s < ZD_CHUNKS)
    def _():
        pltpu.sync_copy(accd.at[pl.ds(s * CH, CH)],
                        den_hbm.at[pl.ds(s * CH, CH), pl.ds(c * 128, 128)])


def _scatter(payn, payd, seg, seg16):
    return pl.kernel(
        _scatter_body,
        out_type=[
            jax.ShapeDtypeStruct((NPAD, 2 * 128), jnp.float32),
            jax.ShapeDtypeStruct((DROWS, 2 * 128), jnp.float32),
        ],
        mesh=_sc_mesh(),
        scratch_types=[
            pltpu.VMEM_SHARED((NPAD, 128), jnp.float32),
            pltpu.VMEM_SHARED((DROWS, 128), jnp.float32),
            pltpu.VMEM((CH, 128), jnp.float32),
            pltpu.VMEM((CH, 128), jnp.float32),
            pltpu.VMEM((CH,), jnp.int32),
            pltpu.VMEM((CH,), jnp.int32),
            pltpu.SemaphoreType.DMA,
            pltpu.SemaphoreType.DMA,
        ],
        compiler_params=_SC_PARAMS,
    )(payn, payd, seg, seg16)


NB = 2048


def _div_body(num_ref, den_ref, out_ref):
    num = num_ref[...]
    den = den_ref[...]
    outs = []
    for h in range(HEADS):
        d = den[:, h:h + 1]
        recip = jnp.where(d != 0.0, 1.0 / d, 0.0)
        outs.append(num[:, h * OUT_DIM:(h + 1) * OUT_DIM] * recip)
    out_ref[...] = jnp.concatenate(outs, axis=1)


def _divide(num, den16):
    return pl.pallas_call(
        _div_body,
        grid=(NPAD // NB,),
        in_specs=[
            pl.BlockSpec((NB, HEADS * OUT_DIM), lambda i: (i, 0)),
            pl.BlockSpec((NB, 8), lambda i: (i, 0)),
        ],
        out_specs=pl.BlockSpec((NB, HEADS * OUT_DIM), lambda i: (i, 0)),
        out_shape=jax.ShapeDtypeStruct((N, HEADS * OUT_DIM), jnp.float32),
    )(num, den16)


def kernel(x, path_list, W_ih, W_hh, b_ih, b_hh, a):
    idx_tm = path_list.T.reshape(-1).astype(jnp.int32)
    seg = path_list[:, L - 1].astype(jnp.int32)
    seg16 = seg // 16
    m16 = (seg % 16).astype(jnp.float32).reshape(P, 1)
    abig = jnp.repeat(a, OUT_DIM, axis=1)
    irep = jnp.tile(jnp.eye(OUT_DIM, dtype=jnp.float32), (1, HEADS))
    apat = jnp.tile(
        jnp.concatenate([a, jnp.zeros((OUT_DIM, 8 - HEADS), jnp.float32)],
                        axis=1), (1, 16))
    feats = _gather(x, idx_tm)
    payn, payd = _gru_payload(
        feats, W_ih.T, W_hh.T, b_ih.reshape(1, -1), b_hh.reshape(1, -1),
        abig, irep, apat, m16)
    num, den = _scatter(payn, payd, seg, seg16)
    den8 = (den[:, :128] + den[:, 128:]).reshape(NPAD, 8)
    return _divide(num, den8)


# speedup vs baseline: 6.0262x; 1.0239x over previous
import jax
import jax.numpy as jnp
from jax import lax
from jax.experimental import pallas as pl
from jax.experimental.pallas import tpu as pltpu
from jax.experimental.pallas import tpu_sc as plsc

N = 10000
NPAD = 10240
P = 80000
L = 4
IN_DIM = 128
OUT_DIM = 64
HEADS = 4

NC = 2
NS = 16
NW = NC * NS

DROWS = NPAD // 16


def _sc_mesh():
    return plsc.VectorSubcoreMesh(
        core_axis_name="c", subcore_axis_name="s", num_cores=NC, num_subcores=NS)


_SC_PARAMS = pltpu.CompilerParams(use_tc_tiling_on_sc=True)

G_ROWS = P // (NW // L)
G_CHUNK = 192
G_ITERS = 52
G_TAIL = G_ROWS - G_ITERS * G_CHUNK
XL_CHUNK = 200
XL_CHUNKS = N // XL_CHUNK


def _gather_body(x_hbm, idx_hbm, out_hbm, xs, idx0, idx1, rows0, rows1,
                 semi0, semi1, semg0, semg1):
    c = lax.axis_index("c")
    s = lax.axis_index("s")
    wid = s * NC + c
    t = wid % L
    start = (wid // L) * G_ROWS

    for j in range((XL_CHUNKS + NS - 1) // NS):
        chunk = s + j * NS

        @pl.when(chunk < XL_CHUNKS)
        def _():
            pltpu.sync_copy(x_hbm.at[pl.ds(chunk * XL_CHUNK, XL_CHUNK)],
                            xs.at[pl.ds(chunk * XL_CHUNK, XL_CHUNK)])
    plsc.subcore_barrier()

    def isrc(j):
        return idx_hbm.at[pl.ds(t * P + start + j * G_CHUNK, G_CHUNK)]

    def istart(j, ib, semi):
        @pl.when(j < G_ITERS)
        def _():
            pltpu.async_copy(isrc(j), ib, semi)

    def iwait(j, ib, semi):
        pltpu.make_async_copy(isrc(j), ib, semi).wait()

    def gstart(ib, rb, semg):
        pltpu.async_copy(xs.at[ib], rb, semg)

    def gwait(ib, rb, semg):
        pltpu.make_async_copy(xs.at[ib], rb, semg).wait()

    def store(j, rb):
        pltpu.sync_copy(rb, out_hbm.at[t, pl.ds(start + j * G_CHUNK, G_CHUNK)])

    istart(0, idx0, semi0)
    iwait(0, idx0, semi0)
    gstart(idx0, rows0, semg0)
    istart(1, idx1, semi1)

    def pair(k, _):
        j1 = 2 * k + 1
        iwait(j1, idx1, semi1)
        gstart(idx1, rows1, semg1)
        gwait(idx0, rows0, semg0)
        istart(j1 + 1, idx0, semi0)
        store(j1 - 1, rows0)
        j2 = 2 * k + 2
        iwait(j2, idx0, semi0)
        gstart(idx0, rows0, semg0)
        gwait(idx1, rows1, semg1)
        istart(j2 + 1, idx1, semi1)
        store(j2 - 1, rows1)
        return ()

    lax.fori_loop(0, (G_ITERS - 2) // 2, pair, ())
    j1 = G_ITERS - 1
    iwait(j1, idx1, semi1)
    gstart(idx1, rows1, semg1)
    gwait(idx0, rows0, semg0)
    store(j1 - 1, rows0)
    gwait(idx1, rows1, semg1)
    store(j1, rows1)
    tb = start + G_ITERS * G_CHUNK
    pltpu.sync_copy(idx_hbm.at[pl.ds(t * P + tb, G_TAIL)], idx0.at[pl.ds(0, G_TAIL)])
    pltpu.async_copy(xs.at[idx0.at[pl.ds(0, G_TAIL)]], rows0.at[pl.ds(0, G_TAIL)], semg0).wait()
    pltpu.sync_copy(rows0.at[pl.ds(0, G_TAIL)], out_hbm.at[t, pl.ds(tb, G_TAIL)])


def _gather(x, idx_tm):
    return pl.kernel(
        _gather_body,
        out_type=jax.ShapeDtypeStruct((L, P, IN_DIM), jnp.float32),
        mesh=_sc_mesh(),
        scratch_types=[
            pltpu.VMEM_SHARED((N, IN_DIM), jnp.float32),
            pltpu.VMEM((G_CHUNK,), jnp.int32),
            pltpu.VMEM((G_CHUNK,), jnp.int32),
            pltpu.VMEM((G_CHUNK, IN_DIM), jnp.float32),
            pltpu.VMEM((G_CHUNK, IN_DIM), jnp.float32),
            pltpu.SemaphoreType.DMA,
            pltpu.SemaphoreType.DMA,
            pltpu.SemaphoreType.DMA,
            pltpu.SemaphoreType.DMA,
        ],
        compiler_params=_SC_PARAMS,
    )(x, idx_tm)


PB = 4000


def _gru_body(f_ref, wi_ref, wh_ref, bi_ref, bh_ref, abig_ref, irep_ref,
              apat_ref, m8_ref, payn_ref, payd_ref):
    def dot(u, v):
        return jnp.dot(u, v, preferred_element_type=jnp.float32)

    wi = wi_ref[...].astype(jnp.bfloat16)
    wh = wh_ref[...]
    b = bi_ref[...] + bh_ref[...]
    h = jnp.zeros((PB, OUT_DIM), jnp.float32)
    for t in range(L):
        xt = f_ref[t].astype(jnp.bfloat16)
        gh = dot(h, wh)
        s = dot(xt, wi) + gh + b
        rz = jax.nn.sigmoid(s[:, :2 * OUT_DIM])
        r = rz[:, :OUT_DIM]
        z = rz[:, OUT_DIM:]
        n = jnp.tanh(s[:, 2 * OUT_DIM:] + (r - 1.0) * (gh[:, 2 * OUT_DIM:] + bh_ref[:, 2 * OUT_DIM:]))
        h = n + z * (h - n)
    payn_ref[...] = jnp.exp(dot(h, abig_ref[...])) * dot(h, irep_ref[...])
    lane = lax.broadcasted_iota(jnp.int32, (PB, 128), 1)
    grp = (lane // 8).astype(jnp.float32)
    mask = ((grp == m8_ref[...]) & (lane % 8 < HEADS)).astype(jnp.float32)
    payd_ref[...] = jnp.exp(dot(h, apat_ref[...])) * mask


def _gru_payload(feats, wi, wh, bi, bh, abig, irep, apat, m8):
    return pl.pallas_call(
        _gru_body,
        grid=(P // PB,),
        in_specs=[
            pl.BlockSpec((L, PB, IN_DIM), lambda i: (0, i, 0)),
            pl.BlockSpec((IN_DIM, 3 * OUT_DIM), lambda i: (0, 0)),
            pl.BlockSpec((OUT_DIM, 3 * OUT_DIM), lambda i: (0, 0)),
            pl.BlockSpec((1, 3 * OUT_DIM), lambda i: (0, 0)),
            pl.BlockSpec((1, 3 * OUT_DIM), lambda i: (0, 0)),
            pl.BlockSpec((OUT_DIM, HEADS * OUT_DIM), lambda i: (0, 0)),
            pl.BlockSpec((OUT_DIM, HEADS * OUT_DIM), lambda i: (0, 0)),
            pl.BlockSpec((OUT_DIM, 128), lambda i: (0, 0)),
            pl.BlockSpec((PB, 1), lambda i: (i, 0)),
        ],
        out_specs=[
            pl.BlockSpec((PB, HEADS * OUT_DIM), lambda i: (i, 0)),
            pl.BlockSpec((PB, 128), lambda i: (i, 0)),
        ],
        out_shape=[
            jax.ShapeDtypeStruct((P, HEADS * OUT_DIM), jnp.float32),
            jax.ShapeDtypeStruct((P, 128), jnp.float32),
        ],
    )(feats, wi, wh, bi, bh, abig, irep, apat, m8)


CH = 160
NCHUNK = P // CH
DCHUNK = P // NC // CH
ZN_CHUNKS = NPAD // CH
ZD_CHUNKS = DROWS // CH


def _scatter_body(payn_hbm, payd_hbm, seg_hbm, seg16_hbm, num_hbm, den_hbm,
                  accn, accd, pbuf0, pbuf1, sbuf0, sbuf1, sem0, sem1):
    c = lax.axis_index("c")
    s = lax.axis_index("s")

    def zrow(i, _):
        for j in range(128 // 16):
            pbuf0[i, pl.ds(j * 16, 16)] = jnp.zeros((16,), jnp.float32)
        return ()

    lax.fori_loop(0, CH, zrow, ())
    for j in range(ZN_CHUNKS // NS):
        pltpu.sync_copy(pbuf0, accn.at[pl.ds((s + j * NS) * CH, CH)])

    @pl.when(s < ZD_CHUNKS)
    def _():
        pltpu.sync_copy(pbuf0, accd.at[pl.ds(s * CH, CH)])
    plsc.subcore_barrier()

    def n_src(j):
        base = (s + j * NS) * CH
        return (seg_hbm.at[pl.ds(base, CH)],
                payn_hbm.at[pl.ds(base, CH), pl.ds(c * 128, 128)])

    def n_start(j, pb, sb, sem):
        @pl.when(s + j * NS < NCHUNK)
        def _():
            sg, pay = n_src(j)
            pltpu.async_copy(sg, sb, sem)
            pltpu.async_copy(pay, pb, sem)

    def n_consume(j, pb, sb, sem):
        @pl.when(s + j * NS < NCHUNK)
        def _():
            sg, pay = n_src(j)
            pltpu.make_async_copy(sg, sb, sem).wait()
            pltpu.make_async_copy(pay, pb, sem).wait()
            pltpu.sync_copy(pb, accn.at[sb], add=True)

    NJ = (NCHUNK + NS - 1) // NS

    n_start(0, pbuf0, sbuf0, sem0)

    def npair(k, _):
        j0 = 2 * k
        n_start(j0 + 1, pbuf1, sbuf1, sem1)
        n_consume(j0, pbuf0, sbuf0, sem0)
        n_start(j0 + 2, pbuf0, sbuf0, sem0)
        n_consume(j0 + 1, pbuf1, sbuf1, sem1)
        return ()

    lax.fori_loop(0, NJ // 2, npair, ())

    def d_src(j):
        base = c * (P // NC) + (s + j * NS) * CH
        return (seg16_hbm.at[pl.ds(base, CH)], payd_hbm.at[pl.ds(base, CH)])

    def d_start(j, pb, sb, sem):
        @pl.when(s + j * NS < DCHUNK)
        def _():
            sg, pay = d_src(j)
            pltpu.async_copy(sg, sb, sem)
            pltpu.async_copy(pay, pb, sem)

    def d_consume(j, pb, sb, sem):
        @pl.when(s + j * NS < DCHUNK)
        def _():
            sg, pay = d_src(j)
            pltpu.make_async_copy(sg, sb, sem).wait()
            pltpu.make_async_copy(pay, pb, sem).wait()
            pltpu.sync_copy(pb, accd.at[sb], add=True)

    NJD = (DCHUNK + NS - 1) // NS

    d_start(0, pbuf0, sbuf0, sem0)

    def dpair(k, _):
        j0 = 2 * k
        d_start(j0 + 1, pbuf1, sbuf1, sem1)
        d_consume(j0, pbuf0, sbuf0, sem0)
        d_start(j0 + 2, pbuf0, sbuf0, sem0)
        d_consume(j0 + 1, pbuf1, sbuf1, sem1)
        return ()

    lax.fori_loop(0, NJD // 2, dpair, ())
    plsc.subcore_barrier()

    for j in range(ZN_CHUNKS // NS):
        base = (s + j * NS) * CH
        pltpu.sync_copy(accn.at[pl.ds(base, CH)],
                        num_hbm.at[pl.ds(base, CH), pl.ds(c * 128, 128)])

    @pl.when(s < ZD_CHUNKS)
    def _():
        pltpu.sync_copy(accd.at[pl.ds(s * CH, CH)],
                        den_hbm.at[pl.ds(s * CH, CH), pl.ds(c * 128, 128)])


def _scatter(payn, payd, seg, seg16):
    return pl.kernel(
        _scatter_body,
        out_type=[
            jax.ShapeDtypeStruct((NPAD, 2 * 128), jnp.float32),
            jax.ShapeDtypeStruct((DROWS, 2 * 128), jnp.float32),
        ],
        mesh=_sc_mesh(),
        scratch_types=[
            pltpu.VMEM_SHARED((NPAD, 128), jnp.float32),
            pltpu.VMEM_SHARED((DROWS, 128), jnp.float32),
            pltpu.VMEM((CH, 128), jnp.float32),
            pltpu.VMEM((CH, 128), jnp.float32),
            pltpu.VMEM((CH,), jnp.int32),
            pltpu.VMEM((CH,), jnp.int32),
            pltpu.SemaphoreType.DMA,
            pltpu.SemaphoreType.DMA,
        ],
        compiler_params=_SC_PARAMS,
    )(payn, payd, seg, seg16)


NB = 2048


def _div_body(num_ref, den_ref, out_ref):
    num = num_ref[...]
    den = den_ref[...]
    outs = []
    for h in range(HEADS):
        d = den[:, h:h + 1]
        recip = jnp.where(d != 0.0, 1.0 / d, 0.0)
        outs.append(num[:, h * OUT_DIM:(h + 1) * OUT_DIM] * recip)
    out_ref[...] = jnp.concatenate(outs, axis=1)


def _divide(num, den16):
    return pl.pallas_call(
        _div_body,
        grid=(NPAD // NB,),
        in_specs=[
            pl.BlockSpec((NB, HEADS * OUT_DIM), lambda i: (i, 0)),
            pl.BlockSpec((NB, 8), lambda i: (i, 0)),
        ],
        out_specs=pl.BlockSpec((NB, HEADS * OUT_DIM), lambda i: (i, 0)),
        out_shape=jax.ShapeDtypeStruct((N, HEADS * OUT_DIM), jnp.float32),
    )(num, den16)


def kernel(x, path_list, W_ih, W_hh, b_ih, b_hh, a):
    idx_tm = path_list.T.reshape(-1).astype(jnp.int32)
    seg = path_list[:, L - 1].astype(jnp.int32)
    seg16 = seg // 16
    m16 = (seg % 16).astype(jnp.float32).reshape(P, 1)
    abig = jnp.repeat(a, OUT_DIM, axis=1)
    irep = jnp.tile(jnp.eye(OUT_DIM, dtype=jnp.float32), (1, HEADS))
    apat = jnp.tile(
        jnp.concatenate([a, jnp.zeros((OUT_DIM, 8 - HEADS), jnp.float32)],
                        axis=1), (1, 16))
    feats = _gather(x, idx_tm)
    payn, payd = _gru_payload(
        feats, W_ih.T, W_hh.T, b_ih.reshape(1, -1), b_hh.reshape(1, -1),
        abig, irep, apat, m16)
    num, den = _scatter(payn, payd, seg, seg16)
    den8 = (den[:, :128] + den[:, 128:]).reshape(NPAD, 8)
    return _divide(num, den8)
